# Initial kernel scaffold; baseline (speedup 1.0000x reference)
import jax
import jax.numpy as jnp
from jax.experimental import pallas as pl


def kernel(x, edge_index, edge_attr, W_enc, b_enc, W_e, b_e, W_n, b_n, ln_ng, ln_nb, ln_eg, ln_eb):
    raise NotImplementedError("write your pallas kernel here")


# speedup vs baseline: 2.0755x; 2.0755x over previous
import functools

import jax
import jax.numpy as jnp
from jax import lax
from jax.experimental import pallas as pl
from jax.experimental.pallas import tpu as pltpu
from jax.experimental.pallas import tpu_sc as plsc

N = 10000
E = 320000
D = 128
DH = 64

NC = 2
NS = 16
NW = NC * NS
IW = 128
NROW = E // IW
RPT = N // NS

_SC_PARAMS = pltpu.CompilerParams(use_tc_tiling_on_sc=False)


@functools.lru_cache(maxsize=None)
def _sc_mesh():
    return plsc.VectorSubcoreMesh(core_axis_name="c", subcore_axis_name="s",
                                  num_cores=NC, num_subcores=NS)


def _worker_rows():
    w = lax.axis_index("s") * NC + lax.axis_index("c")
    rs = (w * (NROW // 4)) // (NW // 4)
    re = ((w + 1) * (NROW // 4)) // (NW // 4)
    return rs, re


def _g2a_body(tp, ia, ib, out, ia_v, ib_v, bufa, bufb, obuf, sema, semb):
    rs, re = _worker_rows()

    def row(r, carry):
        pltpu.sync_copy(ia.at[pl.ds(r, 1)], ia_v)
        pltpu.sync_copy(ib.at[pl.ds(r, 1)], ib_v)
        cpa = pltpu.async_copy(tp.at[ia_v.at[0]], bufa, sema)
        cpb = pltpu.async_copy(tp.at[ib_v.at[0]], bufb, semb)
        cpa.wait()
        cpb.wait()

        def add_row(jj, carry2):
            for k in range(DH // 16):
                obuf[jj, pl.ds(k * 16, 16)] = (
                    bufa[jj, pl.ds(k * 16, 16)]
                    + bufb[jj, pl.ds(DH + k * 16, 16)])
            return carry2

        lax.fori_loop(0, IW, add_row, 0, unroll=4)
        pltpu.sync_copy(obuf, out.at[pl.ds(r * IW, IW)])
        return carry

    lax.fori_loop(rs, re, row, 0)


@functools.lru_cache(maxsize=None)
def _gather2add_kernel():
    return pl.kernel(
        _g2a_body,
        out_type=jax.ShapeDtypeStruct((E, D), jnp.float32),
        mesh=_sc_mesh(),
        compiler_params=_SC_PARAMS,
        scratch_types=[
            pltpu.VMEM((1, IW), jnp.int32),
            pltpu.VMEM((1, IW), jnp.int32),
            pltpu.VMEM((IW, D), jnp.float32),
            pltpu.VMEM((IW, D), jnp.float32),
            pltpu.VMEM((IW, D), jnp.float32),
            pltpu.SemaphoreType.DMA,
            pltpu.SemaphoreType.DMA,
        ],
    )


def _gather2add(p, src, dst):
    return _gather2add_kernel()(p, src, dst)


STG = 125


def _scat_body(rows_hbm, idx_hbm, out, acc_sh, stage, idx_v, rows_v, sem):
    cid = lax.axis_index("c")
    sid = lax.axis_index("s")

    def zrow(j, carry):
        for k in range(D // 16):
            stage[j, pl.ds(k * 16, 16)] = jnp.zeros((16,), jnp.float32)
        return carry

    lax.fori_loop(0, STG, zrow, 0, unroll=4)
    for i in range(RPT // STG):
        pltpu.sync_copy(stage, acc_sh.at[pl.ds(sid * RPT + i * STG, STG)])
    plsc.subcore_barrier()

    rs, re = _worker_rows()

    def row(r, carry):
        pltpu.sync_copy(idx_hbm.at[pl.ds(r, 1)], idx_v)
        pltpu.async_copy(rows_hbm.at[pl.ds(r * IW, IW)], rows_v, sem).wait()
        pltpu.sync_copy(rows_v, acc_sh.at[idx_v.at[0]], add=True)
        return carry

    lax.fori_loop(rs, re, row, 0)
    plsc.subcore_barrier()
    for i in range(RPT // STG):
        pltpu.sync_copy(acc_sh.at[pl.ds(sid * RPT + i * STG, STG)], stage)
        pltpu.sync_copy(stage, out.at[cid, pl.ds(sid * RPT + i * STG, STG)])


@functools.lru_cache(maxsize=None)
def _scatter_kernel():
    return pl.kernel(
        _scat_body,
        out_type=jax.ShapeDtypeStruct((NC, N, D), jnp.float32),
        mesh=_sc_mesh(),
        compiler_params=_SC_PARAMS,
        scratch_types=[
            pltpu.VMEM_SHARED((N, D), jnp.float32),
            pltpu.VMEM((STG, D), jnp.float32),
            pltpu.VMEM((1, IW), jnp.int32),
            pltpu.VMEM((IW, D), jnp.float32),
            pltpu.SemaphoreType.DMA,
        ],
    )


def _scatter_partial(rows, dst):
    return _scatter_kernel()(rows, dst)


BN = 2000
BE = 4000


def _dot(a, b):
    return jnp.dot(a, b, preferred_element_type=jnp.float32)


def _ln(v, g, b):
    m = jnp.mean(v, axis=-1, keepdims=True)
    var = jnp.mean((v - m) ** 2, axis=-1, keepdims=True)
    return (v - m) * jax.lax.rsqrt(var + 1e-5) * g + b


def _proj2_body(x_ref, wa, wb, oa, ob):
    xv = x_ref[...]
    oa[...] = _dot(xv, wa[...])
    ob[...] = _dot(xv, wb[...])


def _proj2(x, wa, wb):
    g = N // BN
    spec_w = pl.BlockSpec((D, 2 * DH), lambda i: (0, 0))
    spec_o = pl.BlockSpec((BN, 2 * DH), lambda i: (i, 0))
    return pl.pallas_call(
        _proj2_body,
        grid=(g,),
        in_specs=[pl.BlockSpec((BN, D), lambda i: (i, 0)), spec_w, spec_w],
        out_specs=[spec_o, spec_o],
        out_shape=[jax.ShapeDtypeStruct((N, 2 * DH), jnp.float32)] * 2,
    )(x, wa, wb)


def _pad_msg(enew):
    be = enew.shape[0]
    return jnp.concatenate(
        [enew, jnp.ones((be, 1), jnp.float32),
         jnp.zeros((be, D - DH - 1), jnp.float32)], axis=1)


def _edge_enc_body(g_ref, ea_ref, w_ref, b_ref, o_ref):
    o_ref[...] = jnp.maximum(
        g_ref[:, :DH] + _dot(ea_ref[...], w_ref[...]) + b_ref[...], 0.0)


def _edge_enc(g, edge_attr, w, b):
    de = edge_attr.shape[1]
    return pl.pallas_call(
        _edge_enc_body,
        grid=(E // BE,),
        in_specs=[
            pl.BlockSpec((BE, D), lambda i: (i, 0)),
            pl.BlockSpec((BE, de), lambda i: (i, 0)),
            pl.BlockSpec((de, DH), lambda i: (0, 0)),
            pl.BlockSpec((1, DH), lambda i: (0, 0)),
        ],
        out_specs=pl.BlockSpec((BE, DH), lambda i: (i, 0)),
        out_shape=jax.ShapeDtypeStruct((E, DH), jnp.float32),
    )(g, edge_attr, w, b)


def _edge_iter_body(g_ref, ep_ref, einit_ref, w_ref, b_ref, lg_ref, lb_ref,
                    onew_ref, oln_ref):
    enew = jnp.maximum(
        g_ref[:, :DH] + _dot(ep_ref[...], w_ref[...]) + b_ref[...], 0.0)
    onew_ref[...] = _pad_msg(enew)
    oln_ref[...] = _ln(einit_ref[...] + enew, lg_ref[...], lb_ref[...])


def _edge_iter(g, eprev, einit, w, b, lg, lb):
    bh = pl.BlockSpec((BE, DH), lambda i: (i, 0))
    row_h = pl.BlockSpec((1, DH), lambda i: (0, 0))
    return pl.pallas_call(
        _edge_iter_body,
        grid=(E // BE,),
        in_specs=[pl.BlockSpec((BE, D), lambda i: (i, 0)), bh, bh,
                  pl.BlockSpec((DH, DH), lambda i: (0, 0)),
                  row_h, row_h, row_h],
        out_specs=[pl.BlockSpec((BE, D), lambda i: (i, 0)), bh],
        out_shape=[jax.ShapeDtypeStruct((E, D), jnp.float32),
                   jax.ShapeDtypeStruct((E, DH), jnp.float32)],
    )(g, eprev, einit, w, b, lg, lb)


def _node_body(np_ref, x_ref, agg_ref, wn_ref, wa_ref, b_ref,
               lg_ref, lb_ref, *rest):
    agg = agg_ref[0, :, :DH] + agg_ref[1, :, :DH]
    c = agg_ref[0, :, DH:DH + 1] + agg_ref[1, :, DH:DH + 1]
    inv = 1.0 / jnp.maximum(c, 1.0)
    h = jnp.maximum(
        _dot(np_ref[...], wn_ref[...]) + _dot(agg * inv, wa_ref[...])
        + b_ref[...], 0.0)
    node = _ln(x_ref[...] + h, lg_ref[...], lb_ref[...])
    if len(rest) == 1:
        rest[0][...] = node
    else:
        wp_ref, onode, op_ref = rest
        onode[...] = node
        op_ref[...] = _dot(node, wp_ref[...])


def _node_update(nprev, x, agg_parts, wn, wa, b, lg, lb, wp=None):
    g = N // BN
    bn_d = pl.BlockSpec((BN, D), lambda i: (i, 0))
    row_d = pl.BlockSpec((1, D), lambda i: (0, 0))
    in_specs = [bn_d, bn_d,
                pl.BlockSpec((NC, BN, D), lambda i: (0, i, 0)),
                pl.BlockSpec((D, D), lambda i: (0, 0)),
                pl.BlockSpec((DH, D), lambda i: (0, 0)),
                row_d, row_d, row_d]
    args = [nprev, x, agg_parts, wn, wa, b, lg, lb]
    if wp is None:
        out_specs = bn_d
        out_shape = jax.ShapeDtypeStruct((N, D), jnp.float32)
    else:
        in_specs += [pl.BlockSpec((D, 2 * DH), lambda i: (0, 0))]
        args += [wp]
        out_specs = [bn_d, pl.BlockSpec((BN, 2 * DH), lambda i: (i, 0))]
        out_shape = [jax.ShapeDtypeStruct((N, D), jnp.float32),
                     jax.ShapeDtypeStruct((N, 2 * DH), jnp.float32)]
    return pl.pallas_call(
        _node_body,
        grid=(g,),
        in_specs=in_specs,
        out_specs=out_specs,
        out_shape=out_shape,
    )(*args)


def kernel(x, edge_index, edge_attr, W_enc, b_enc, W_e, b_e, W_n, b_n,
           ln_ng, ln_nb, ln_eg, ln_eb):
    src = edge_index[0].reshape(NROW, IW)
    dst = edge_index[1].reshape(NROW, IW)

    W_enc_s, W_enc_t, W_enc_e = W_enc[:D], W_enc[D:2 * D], W_enc[2 * D:]
    We_s, We_t, We_e = W_e[:D], W_e[D:2 * D], W_e[2 * D:]
    Wn_n, Wn_a = W_n[:D], W_n[D:]
    b_enc2 = b_enc.reshape(1, DH)
    b_e2 = b_e.reshape(1, DH)
    b_n2 = b_n.reshape(1, D)
    ln_eg2, ln_eb2 = ln_eg.reshape(1, DH), ln_eb.reshape(1, DH)
    ln_ng2, ln_nb2 = ln_ng.reshape(1, D), ln_nb.reshape(1, D)

    Wcat_enc = jnp.concatenate([W_enc_s, W_enc_t], axis=1)
    Wcat_e = jnp.concatenate([We_s, We_t], axis=1)

    p_enc, p1 = _proj2(x, Wcat_enc, Wcat_e)

    g_enc = _gather2add(p_enc, src, dst)
    g1 = _gather2add(p1, src, dst)

    init_edge = _edge_enc(g_enc, edge_attr, W_enc_e, b_enc2)

    e_new1, e_ln1 = _edge_iter(g1, init_edge, init_edge, We_e, b_e2,
                               ln_eg2, ln_eb2)
    agg1 = _scatter_partial(e_new1, dst)
    node1, p2 = _node_update(x, x, agg1, Wn_n, Wn_a, b_n2,
                             ln_ng2, ln_nb2, Wcat_e)

    g2 = _gather2add(p2, src, dst)
    e_new2, e_ln2 = _edge_iter(g2, e_ln1, init_edge, We_e, b_e2,
                               ln_eg2, ln_eb2)
    agg2 = _scatter_partial(e_new2, dst)
    node2 = _node_update(node1, x, agg2, Wn_n, Wn_a, b_n2, ln_ng2, ln_nb2)

    return node2, e_ln2


# speedup vs baseline: 2.9971x; 1.4440x over previous
import functools

import jax
import jax.numpy as jnp
from jax import lax
from jax.experimental import pallas as pl
from jax.experimental.pallas import tpu as pltpu
from jax.experimental.pallas import tpu_sc as plsc

N = 10000
E = 320000
D = 128
DH = 64

NC = 2
NS = 16
NW = NC * NS
IW = 128
NROW = E // IW
RPT = N // NS
GRPW = 80
EPAD = GRPW * NW * IW
GBLK = 8
NGBLK = GRPW // GBLK

_SC_PARAMS = pltpu.CompilerParams(use_tc_tiling_on_sc=False)


@functools.lru_cache(maxsize=None)
def _sc_mesh():
    return plsc.VectorSubcoreMesh(core_axis_name="c", subcore_axis_name="s",
                                  num_cores=NC, num_subcores=NS)


def _worker_rows():
    w = lax.axis_index("s") * NC + lax.axis_index("c")
    rs = (w * (NROW // 4)) // (NW // 4)
    re = ((w + 1) * (NROW // 4)) // (NW // 4)
    return rs, re


def _gather_pipelined(ts, td, ia, ib, out, ia_v, ib_v, a0, b0, a1, b1,
                      o0, o1, sema, semb, semo, dual):
    w = lax.axis_index("s") * NC + lax.axis_index("c")
    base = w * GRPW
    bufs = [(a0, b0), (a1, b1)]
    obufs = [o0, o1]

    def block(blk, carry):
        r = base + blk * GBLK
        pltpu.sync_copy(ia.at[pl.ds(r, GBLK)], ia_v)
        pltpu.sync_copy(ib.at[pl.ds(r, GBLK)], ib_v)

        def fire(j):
            aa, bb = bufs[j % 2]
            return (pltpu.async_copy(ts.at[ia_v.at[j]], aa, sema),
                    pltpu.async_copy(td.at[ib_v.at[j]], bb, semb))

        cur = fire(0)
        owaits = [None, None]
        for j in range(GBLK):
            nxt = fire(j + 1) if j < GBLK - 1 else None
            cur[0].wait()
            cur[1].wait()
            aa, bb = bufs[j % 2]
            ob = obufs[j % 2]
            if owaits[j % 2] is not None:
                owaits[j % 2].wait()

            if dual:
                def add_row(jj, c2, aa=aa, bb=bb, ob=ob):
                    for k in range(D // 16):
                        s = pl.ds(k * 16, 16)
                        ob[jj, s] = aa[jj, s] + bb[jj, s]
                    return c2
            else:
                def add_row(jj, c2, aa=aa, bb=bb, ob=ob):
                    for k in range(DH // 16):
                        ob[jj, pl.ds(k * 16, 16)] = (
                            aa[jj, pl.ds(k * 16, 16)]
                            + bb[jj, pl.ds(DH + k * 16, 16)])
                    return c2

            lax.fori_loop(0, IW, add_row, 0, unroll=4)
            owaits[j % 2] = pltpu.async_copy(
                ob, out.at[pl.ds((r + j) * IW, IW)], semo)
            cur = nxt
        for ow in owaits:
            if ow is not None:
                ow.wait()
        return carry

    lax.fori_loop(0, NGBLK, block, 0)


_GATHER_SCRATCH = [
    pltpu.VMEM((GBLK, IW), jnp.int32),
    pltpu.VMEM((GBLK, IW), jnp.int32),
    pltpu.VMEM((IW, D), jnp.float32),
    pltpu.VMEM((IW, D), jnp.float32),
    pltpu.VMEM((IW, D), jnp.float32),
    pltpu.VMEM((IW, D), jnp.float32),
    pltpu.VMEM((IW, D), jnp.float32),
    pltpu.VMEM((IW, D), jnp.float32),
    pltpu.SemaphoreType.DMA,
    pltpu.SemaphoreType.DMA,
    pltpu.SemaphoreType.DMA,
]


@functools.lru_cache(maxsize=None)
def _gather_dual_kernel():
    body = functools.partial(_gather_pipelined, dual=True)
    return pl.kernel(
        lambda ts, td, ia, ib, out, *s: body(ts, td, ia, ib, out, *s),
        out_type=jax.ShapeDtypeStruct((EPAD, D), jnp.float32),
        mesh=_sc_mesh(),
        compiler_params=_SC_PARAMS,
        scratch_types=_GATHER_SCRATCH,
    )


@functools.lru_cache(maxsize=None)
def _gather_single_kernel():
    body = functools.partial(_gather_pipelined, dual=False)
    return pl.kernel(
        lambda tp, ia, ib, out, *s: body(tp, tp, ia, ib, out, *s),
        out_type=jax.ShapeDtypeStruct((EPAD, D), jnp.float32),
        mesh=_sc_mesh(),
        compiler_params=_SC_PARAMS,
        scratch_types=_GATHER_SCRATCH,
    )


def _gather_dual(ts, td, src, dst):
    return _gather_dual_kernel()(ts, td, src, dst)


def _gather2add(p, src, dst):
    return _gather_single_kernel()(p, src, dst)


STG = 125


def _scat_body(rows_hbm, idx_hbm, out, acc_sh, stage, idx_v, rows_v, sem):
    cid = lax.axis_index("c")
    sid = lax.axis_index("s")

    def zrow(j, carry):
        for k in range(D // 16):
            stage[j, pl.ds(k * 16, 16)] = jnp.zeros((16,), jnp.float32)
        return carry

    lax.fori_loop(0, STG, zrow, 0, unroll=4)
    for i in range(RPT // STG):
        pltpu.sync_copy(stage, acc_sh.at[pl.ds(sid * RPT + i * STG, STG)])
    plsc.subcore_barrier()

    rs, re = _worker_rows()

    def row(r, carry):
        pltpu.sync_copy(idx_hbm.at[pl.ds(r, 1)], idx_v)
        pltpu.async_copy(rows_hbm.at[pl.ds(r * IW, IW)], rows_v, sem).wait()
        pltpu.sync_copy(rows_v, acc_sh.at[idx_v.at[0]], add=True)
        return carry

    lax.fori_loop(rs, re, row, 0)
    plsc.subcore_barrier()
    for i in range(RPT // STG):
        pltpu.sync_copy(acc_sh.at[pl.ds(sid * RPT + i * STG, STG)], stage)
        pltpu.sync_copy(stage, out.at[cid, pl.ds(sid * RPT + i * STG, STG)])


@functools.lru_cache(maxsize=None)
def _scatter_kernel():
    return pl.kernel(
        _scat_body,
        out_type=jax.ShapeDtypeStruct((NC, N, D), jnp.float32),
        mesh=_sc_mesh(),
        compiler_params=_SC_PARAMS,
        scratch_types=[
            pltpu.VMEM_SHARED((N, D), jnp.float32),
            pltpu.VMEM((STG, D), jnp.float32),
            pltpu.VMEM((1, IW), jnp.int32),
            pltpu.VMEM((IW, D), jnp.float32),
            pltpu.SemaphoreType.DMA,
        ],
    )


def _scatter_partial(rows, dst):
    return _scatter_kernel()(rows, dst)


BN = 2000
BE = 4000


def _dot(a, b):
    return jnp.dot(a, b, preferred_element_type=jnp.float32)


def _ln(v, g, b):
    m = jnp.mean(v, axis=-1, keepdims=True)
    var = jnp.mean((v - m) ** 2, axis=-1, keepdims=True)
    return (v - m) * jax.lax.rsqrt(var + 1e-5) * g + b


def _proj2_body(x_ref, wa, wb, oa, ob):
    xv = x_ref[...]
    oa[...] = _dot(xv, wa[...])
    ob[...] = _dot(xv, wb[...])


def _proj2(x, wa, wb):
    g = N // BN
    spec_w = pl.BlockSpec((D, 2 * DH), lambda i: (0, 0))
    spec_o = pl.BlockSpec((BN, 2 * DH), lambda i: (i, 0))
    return pl.pallas_call(
        _proj2_body,
        grid=(g,),
        in_specs=[pl.BlockSpec((BN, D), lambda i: (i, 0)), spec_w, spec_w],
        out_specs=[spec_o, spec_o],
        out_shape=[jax.ShapeDtypeStruct((N, 2 * DH), jnp.float32)] * 2,
    )(x, wa, wb)


def _pad_msg(enew):
    be = enew.shape[0]
    return jnp.concatenate(
        [enew, jnp.ones((be, 1), jnp.float32),
         jnp.zeros((be, D - DH - 1), jnp.float32)], axis=1)


def _edge_fused_body(g_ref, ea_ref, wenc_ref, benc_ref, we_ref, be_ref,
                     lg_ref, lb_ref, oinit, onew, oln):
    gb = g_ref[...]
    init = jnp.maximum(
        gb[:, :DH] + _dot(ea_ref[...], wenc_ref[...]) + benc_ref[...], 0.0)
    oinit[...] = init
    enew = jnp.maximum(
        gb[:, DH:] + _dot(init, we_ref[...]) + be_ref[...], 0.0)
    onew[...] = _pad_msg(enew)
    oln[...] = _ln(init + enew, lg_ref[...], lb_ref[...])


def _edge_fused(g_both, edge_attr, wenc, benc, we, be, lg, lb):
    de = edge_attr.shape[1]
    bh = pl.BlockSpec((BE, DH), lambda i: (i, 0))
    row_h = pl.BlockSpec((1, DH), lambda i: (0, 0))
    return pl.pallas_call(
        _edge_fused_body,
        grid=(E // BE,),
        in_specs=[
            pl.BlockSpec((BE, D), lambda i: (i, 0)),
            pl.BlockSpec((BE, de), lambda i: (i, 0)),
            pl.BlockSpec((de, DH), lambda i: (0, 0)),
            row_h,
            pl.BlockSpec((DH, DH), lambda i: (0, 0)),
            row_h, row_h, row_h,
        ],
        out_specs=[bh, pl.BlockSpec((BE, D), lambda i: (i, 0)), bh],
        out_shape=[jax.ShapeDtypeStruct((E, DH), jnp.float32),
                   jax.ShapeDtypeStruct((E, D), jnp.float32),
                   jax.ShapeDtypeStruct((E, DH), jnp.float32)],
    )(g_both, edge_attr, wenc, benc, we, be, lg, lb)


def _edge_iter_body(g_ref, ep_ref, einit_ref, w_ref, b_ref, lg_ref, lb_ref,
                    onew_ref, oln_ref):
    enew = jnp.maximum(
        g_ref[:, :DH] + _dot(ep_ref[...], w_ref[...]) + b_ref[...], 0.0)
    onew_ref[...] = _pad_msg(enew)
    oln_ref[...] = _ln(einit_ref[...] + enew, lg_ref[...], lb_ref[...])


def _edge_iter(g, eprev, einit, w, b, lg, lb):
    bh = pl.BlockSpec((BE, DH), lambda i: (i, 0))
    row_h = pl.BlockSpec((1, DH), lambda i: (0, 0))
    return pl.pallas_call(
        _edge_iter_body,
        grid=(E // BE,),
        in_specs=[pl.BlockSpec((BE, D), lambda i: (i, 0)), bh, bh,
                  pl.BlockSpec((DH, DH), lambda i: (0, 0)),
                  row_h, row_h, row_h],
        out_specs=[pl.BlockSpec((BE, D), lambda i: (i, 0)), bh],
        out_shape=[jax.ShapeDtypeStruct((E, D), jnp.float32),
                   jax.ShapeDtypeStruct((E, DH), jnp.float32)],
    )(g, eprev, einit, w, b, lg, lb)


def _node_body(np_ref, x_ref, agg_ref, wn_ref, wa_ref, b_ref,
               lg_ref, lb_ref, *rest):
    agg = agg_ref[0, :, :DH] + agg_ref[1, :, :DH]
    c = agg_ref[0, :, DH:DH + 1] + agg_ref[1, :, DH:DH + 1]
    inv = 1.0 / jnp.maximum(c, 1.0)
    h = jnp.maximum(
        _dot(np_ref[...], wn_ref[...]) + _dot(agg * inv, wa_ref[...])
        + b_ref[...], 0.0)
    node = _ln(x_ref[...] + h, lg_ref[...], lb_ref[...])
    if len(rest) == 1:
        rest[0][...] = node
    else:
        wp_ref, onode, op_ref = rest
        onode[...] = node
        op_ref[...] = _dot(node, wp_ref[...])


def _node_update(nprev, x, agg_parts, wn, wa, b, lg, lb, wp=None):
    g = N // BN
    bn_d = pl.BlockSpec((BN, D), lambda i: (i, 0))
    row_d = pl.BlockSpec((1, D), lambda i: (0, 0))
    in_specs = [bn_d, bn_d,
                pl.BlockSpec((NC, BN, D), lambda i: (0, i, 0)),
                pl.BlockSpec((D, D), lambda i: (0, 0)),
                pl.BlockSpec((DH, D), lambda i: (0, 0)),
                row_d, row_d, row_d]
    args = [nprev, x, agg_parts, wn, wa, b, lg, lb]
    if wp is None:
        out_specs = bn_d
        out_shape = jax.ShapeDtypeStruct((N, D), jnp.float32)
    else:
        in_specs += [pl.BlockSpec((D, 2 * DH), lambda i: (0, 0))]
        args += [wp]
        out_specs = [bn_d, pl.BlockSpec((BN, 2 * DH), lambda i: (i, 0))]
        out_shape = [jax.ShapeDtypeStruct((N, D), jnp.float32),
                     jax.ShapeDtypeStruct((N, 2 * DH), jnp.float32)]
    return pl.pallas_call(
        _node_body,
        grid=(g,),
        in_specs=in_specs,
        out_specs=out_specs,
        out_shape=out_shape,
    )(*args)


def kernel(x, edge_index, edge_attr, W_enc, b_enc, W_e, b_e, W_n, b_n,
           ln_ng, ln_nb, ln_eg, ln_eb):
    src_flat = edge_index[0]
    dst_flat = edge_index[1]
    dst = dst_flat.reshape(NROW, IW)
    pad_idx = (jnp.arange(EPAD - E, dtype=jnp.int32) * 13) % N
    src_p = jnp.concatenate([src_flat, pad_idx]).reshape(EPAD // IW, IW)
    dst_p = jnp.concatenate([dst_flat, pad_idx]).reshape(EPAD // IW, IW)

    W_enc_s, W_enc_t, W_enc_e = W_enc[:D], W_enc[D:2 * D], W_enc[2 * D:]
    We_s, We_t, We_e = W_e[:D], W_e[D:2 * D], W_e[2 * D:]
    Wn_n, Wn_a = W_n[:D], W_n[D:]
    b_enc2 = b_enc.reshape(1, DH)
    b_e2 = b_e.reshape(1, DH)
    b_n2 = b_n.reshape(1, D)
    ln_eg2, ln_eb2 = ln_eg.reshape(1, DH), ln_eb.reshape(1, DH)
    ln_ng2, ln_nb2 = ln_ng.reshape(1, D), ln_nb.reshape(1, D)

    Ws_cat = jnp.concatenate([W_enc_s, We_s], axis=1)
    Wt_cat = jnp.concatenate([W_enc_t, We_t], axis=1)
    Wcat_e = jnp.concatenate([We_s, We_t], axis=1)

    t_src, t_dst = _proj2(x, Ws_cat, Wt_cat)

    g_both = _gather_dual(t_src, t_dst, src_p, dst_p)

    init_edge, e_new1, e_ln1 = _edge_fused(
        g_both, edge_attr, W_enc_e, b_enc2, We_e, b_e2, ln_eg2, ln_eb2)
    agg1 = _scatter_partial(e_new1, dst)
    node1, p2 = _node_update(x, x, agg1, Wn_n, Wn_a, b_n2,
                             ln_ng2, ln_nb2, Wcat_e)

    g2 = _gather2add(p2, src_p, dst_p)
    e_new2, e_ln2 = _edge_iter(g2, e_ln1, init_edge, We_e, b_e2,
                               ln_eg2, ln_eb2)
    agg2 = _scatter_partial(e_new2, dst)
    node2 = _node_update(node1, x, agg2, Wn_n, Wn_a, b_n2, ln_ng2, ln_nb2)

    return node2, e_ln2


# speedup vs baseline: 3.5870x; 1.1968x over previous
import functools

import jax
import jax.numpy as jnp
from jax import lax
from jax.experimental import pallas as pl
from jax.experimental.pallas import tpu as pltpu
from jax.experimental.pallas import tpu_sc as plsc

N = 10000
E = 320000
D = 128
DH = 64

NC = 2
NS = 16
NW = NC * NS
IW = 128
NROW = E // IW
RPT = N // NS
GRPW = 80
EPAD = GRPW * NW * IW
GBLK = 8
NGBLK = GRPW // GBLK

_SC_PARAMS = pltpu.CompilerParams(use_tc_tiling_on_sc=False)


@functools.lru_cache(maxsize=None)
def _sc_mesh():
    return plsc.VectorSubcoreMesh(core_axis_name="c", subcore_axis_name="s",
                                  num_cores=NC, num_subcores=NS)


def _worker_rows():
    w = lax.axis_index("s") * NC + lax.axis_index("c")
    rs = (w * (NROW // 4)) // (NW // 4)
    re = ((w + 1) * (NROW // 4)) // (NW // 4)
    return rs, re


def _gather_pipelined(ts, td, ia, ib, out, ia_v, ib_v, a0, b0, a1, b1,
                      o0, o1, sema, semb, semo, dual):
    w = lax.axis_index("s") * NC + lax.axis_index("c")
    base = w * GRPW
    bufs = [(a0, b0), (a1, b1)]
    obufs = [o0, o1]

    def block(blk, carry):
        r = base + blk * GBLK
        pltpu.sync_copy(ia.at[pl.ds(r, GBLK)], ia_v)
        pltpu.sync_copy(ib.at[pl.ds(r, GBLK)], ib_v)

        def fire(j):
            aa, bb = bufs[j % 2]
            return (pltpu.async_copy(ts.at[ia_v.at[j]], aa, sema),
                    pltpu.async_copy(td.at[ib_v.at[j]], bb, semb))

        cur = fire(0)
        owaits = [None, None]
        for j in range(GBLK):
            nxt = fire(j + 1) if j < GBLK - 1 else None
            cur[0].wait()
            cur[1].wait()
            aa, bb = bufs[j % 2]
            ob = obufs[j % 2]
            if owaits[j % 2] is not None:
                owaits[j % 2].wait()

            if dual:
                @plsc.parallel_loop(0, IW, step=1, unroll=4)
                def add_row(jj, aa=aa, bb=bb, ob=ob):
                    for k in range(D // 16):
                        s = pl.ds(k * 16, 16)
                        ob[jj, s] = aa[jj, s] + bb[jj, s]
            else:
                @plsc.parallel_loop(0, IW, step=1, unroll=4)
                def add_row(jj, aa=aa, bb=bb, ob=ob):
                    for k in range(DH // 16):
                        ob[jj, pl.ds(k * 16, 16)] = (
                            aa[jj, pl.ds(k * 16, 16)]
                            + bb[jj, pl.ds(DH + k * 16, 16)])
            owaits[j % 2] = pltpu.async_copy(
                ob, out.at[pl.ds((r + j) * IW, IW)], semo)
            cur = nxt
        for ow in owaits:
            if ow is not None:
                ow.wait()
        return carry

    lax.fori_loop(0, NGBLK, block, 0)


_GATHER_SCRATCH = [
    pltpu.VMEM((GBLK, IW), jnp.int32),
    pltpu.VMEM((GBLK, IW), jnp.int32),
    pltpu.VMEM((IW, D), jnp.float32),
    pltpu.VMEM((IW, D), jnp.float32),
    pltpu.VMEM((IW, D), jnp.float32),
    pltpu.VMEM((IW, D), jnp.float32),
    pltpu.VMEM((IW, D), jnp.float32),
    pltpu.VMEM((IW, D), jnp.float32),
    pltpu.SemaphoreType.DMA,
    pltpu.SemaphoreType.DMA,
    pltpu.SemaphoreType.DMA,
]


@functools.lru_cache(maxsize=None)
def _gather_dual_kernel():
    body = functools.partial(_gather_pipelined, dual=True)
    return pl.kernel(
        lambda ts, td, ia, ib, out, *s: body(ts, td, ia, ib, out, *s),
        out_type=jax.ShapeDtypeStruct((EPAD, D), jnp.float32),
        mesh=_sc_mesh(),
        compiler_params=_SC_PARAMS,
        scratch_types=_GATHER_SCRATCH,
    )


@functools.lru_cache(maxsize=None)
def _gather_single_kernel():
    body = functools.partial(_gather_pipelined, dual=False)
    return pl.kernel(
        lambda tp, ia, ib, out, *s: body(tp, tp, ia, ib, out, *s),
        out_type=jax.ShapeDtypeStruct((EPAD, D), jnp.float32),
        mesh=_sc_mesh(),
        compiler_params=_SC_PARAMS,
        scratch_types=_GATHER_SCRATCH,
    )


def _gather_dual(ts, td, src, dst):
    return _gather_dual_kernel()(ts, td, src, dst)


def _gather2add(p, src, dst):
    return _gather_single_kernel()(p, src, dst)


STG = 125


def _scat_body(rows_hbm, idx_hbm, out, acc_sh, stage, idx_v, rows_v, sem):
    cid = lax.axis_index("c")
    sid = lax.axis_index("s")

    @plsc.parallel_loop(0, STG, step=1, unroll=4)
    def zrow(j):
        for k in range(D // 16):
            stage[j, pl.ds(k * 16, 16)] = jnp.zeros((16,), jnp.float32)
    for i in range(RPT // STG):
        pltpu.sync_copy(stage, acc_sh.at[pl.ds(sid * RPT + i * STG, STG)])
    plsc.subcore_barrier()

    rs, re = _worker_rows()

    def row(r, carry):
        pltpu.sync_copy(idx_hbm.at[pl.ds(r, 1)], idx_v)
        pltpu.async_copy(rows_hbm.at[pl.ds(r * IW, IW)], rows_v, sem).wait()
        pltpu.sync_copy(rows_v, acc_sh.at[idx_v.at[0]], add=True)
        return carry

    lax.fori_loop(rs, re, row, 0)
    plsc.subcore_barrier()
    for i in range(RPT // STG):
        pltpu.sync_copy(acc_sh.at[pl.ds(sid * RPT + i * STG, STG)], stage)
        pltpu.sync_copy(stage, out.at[cid, pl.ds(sid * RPT + i * STG, STG)])


@functools.lru_cache(maxsize=None)
def _scatter_kernel():
    return pl.kernel(
        _scat_body,
        out_type=jax.ShapeDtypeStruct((NC, N, D), jnp.float32),
        mesh=_sc_mesh(),
        compiler_params=_SC_PARAMS,
        scratch_types=[
            pltpu.VMEM_SHARED((N, D), jnp.float32),
            pltpu.VMEM((STG, D), jnp.float32),
            pltpu.VMEM((1, IW), jnp.int32),
            pltpu.VMEM((IW, D), jnp.float32),
            pltpu.SemaphoreType.DMA,
        ],
    )


def _scatter_partial(rows, dst):
    return _scatter_kernel()(rows, dst)


BN = 2000
BE = 4000


def _dot(a, b):
    return jnp.dot(a, b, preferred_element_type=jnp.float32)


def _ln(v, g, b):
    m = jnp.mean(v, axis=-1, keepdims=True)
    var = jnp.mean((v - m) ** 2, axis=-1, keepdims=True)
    return (v - m) * jax.lax.rsqrt(var + 1e-5) * g + b


def _proj2_body(x_ref, wa, wb, oa, ob):
    xv = x_ref[...]
    oa[...] = _dot(xv, wa[...])
    ob[...] = _dot(xv, wb[...])


def _proj2(x, wa, wb):
    g = N // BN
    spec_w = pl.BlockSpec((D, 2 * DH), lambda i: (0, 0))
    spec_o = pl.BlockSpec((BN, 2 * DH), lambda i: (i, 0))
    return pl.pallas_call(
        _proj2_body,
        grid=(g,),
        in_specs=[pl.BlockSpec((BN, D), lambda i: (i, 0)), spec_w, spec_w],
        out_specs=[spec_o, spec_o],
        out_shape=[jax.ShapeDtypeStruct((N, 2 * DH), jnp.float32)] * 2,
    )(x, wa, wb)


def _pad_msg(enew):
    be = enew.shape[0]
    return jnp.concatenate(
        [enew, jnp.ones((be, 1), jnp.float32),
         jnp.zeros((be, D - DH - 1), jnp.float32)], axis=1)


def _edge_fused_body(g_ref, ea_ref, wenc_ref, benc_ref, we_ref, be_ref,
                     lg_ref, lb_ref, oinit, onew, oln):
    gb = g_ref[...]
    init = jnp.maximum(
        gb[:, :DH] + _dot(ea_ref[...], wenc_ref[...]) + benc_ref[...], 0.0)
    oinit[...] = init
    enew = jnp.maximum(
        gb[:, DH:] + _dot(init, we_ref[...]) + be_ref[...], 0.0)
    onew[...] = _pad_msg(enew)
    oln[...] = _ln(init + enew, lg_ref[...], lb_ref[...])


def _edge_fused(g_both, edge_attr, wenc, benc, we, be, lg, lb):
    de = edge_attr.shape[1]
    bh = pl.BlockSpec((BE, DH), lambda i: (i, 0))
    row_h = pl.BlockSpec((1, DH), lambda i: (0, 0))
    return pl.pallas_call(
        _edge_fused_body,
        grid=(E // BE,),
        in_specs=[
            pl.BlockSpec((BE, D), lambda i: (i, 0)),
            pl.BlockSpec((BE, de), lambda i: (i, 0)),
            pl.BlockSpec((de, DH), lambda i: (0, 0)),
            row_h,
            pl.BlockSpec((DH, DH), lambda i: (0, 0)),
            row_h, row_h, row_h,
        ],
        out_specs=[bh, pl.BlockSpec((BE, D), lambda i: (i, 0)), bh],
        out_shape=[jax.ShapeDtypeStruct((E, DH), jnp.float32),
                   jax.ShapeDtypeStruct((E, D), jnp.float32),
                   jax.ShapeDtypeStruct((E, DH), jnp.float32)],
    )(g_both, edge_attr, wenc, benc, we, be, lg, lb)


def _edge_iter_body(g_ref, ep_ref, einit_ref, w_ref, b_ref, lg_ref, lb_ref,
                    onew_ref, oln_ref):
    enew = jnp.maximum(
        g_ref[:, :DH] + _dot(ep_ref[...], w_ref[...]) + b_ref[...], 0.0)
    onew_ref[...] = _pad_msg(enew)
    oln_ref[...] = _ln(einit_ref[...] + enew, lg_ref[...], lb_ref[...])


def _edge_iter(g, eprev, einit, w, b, lg, lb):
    bh = pl.BlockSpec((BE, DH), lambda i: (i, 0))
    row_h = pl.BlockSpec((1, DH), lambda i: (0, 0))
    return pl.pallas_call(
        _edge_iter_body,
        grid=(E // BE,),
        in_specs=[pl.BlockSpec((BE, D), lambda i: (i, 0)), bh, bh,
                  pl.BlockSpec((DH, DH), lambda i: (0, 0)),
                  row_h, row_h, row_h],
        out_specs=[pl.BlockSpec((BE, D), lambda i: (i, 0)), bh],
        out_shape=[jax.ShapeDtypeStruct((E, D), jnp.float32),
                   jax.ShapeDtypeStruct((E, DH), jnp.float32)],
    )(g, eprev, einit, w, b, lg, lb)


def _node_body(np_ref, x_ref, agg_ref, wn_ref, wa_ref, b_ref,
               lg_ref, lb_ref, *rest):
    agg = agg_ref[0, :, :DH] + agg_ref[1, :, :DH]
    c = agg_ref[0, :, DH:DH + 1] + agg_ref[1, :, DH:DH + 1]
    inv = 1.0 / jnp.maximum(c, 1.0)
    h = jnp.maximum(
        _dot(np_ref[...], wn_ref[...]) + _dot(agg * inv, wa_ref[...])
        + b_ref[...], 0.0)
    node = _ln(x_ref[...] + h, lg_ref[...], lb_ref[...])
    if len(rest) == 1:
        rest[0][...] = node
    else:
        wp_ref, onode, op_ref = rest
        onode[...] = node
        op_ref[...] = _dot(node, wp_ref[...])


def _node_update(nprev, x, agg_parts, wn, wa, b, lg, lb, wp=None):
    g = N // BN
    bn_d = pl.BlockSpec((BN, D), lambda i: (i, 0))
    row_d = pl.BlockSpec((1, D), lambda i: (0, 0))
    in_specs = [bn_d, bn_d,
                pl.BlockSpec((NC, BN, D), lambda i: (0, i, 0)),
                pl.BlockSpec((D, D), lambda i: (0, 0)),
                pl.BlockSpec((DH, D), lambda i: (0, 0)),
                row_d, row_d, row_d]
    args = [nprev, x, agg_parts, wn, wa, b, lg, lb]
    if wp is None:
        out_specs = bn_d
        out_shape = jax.ShapeDtypeStruct((N, D), jnp.float32)
    else:
        in_specs += [pl.BlockSpec((D, 2 * DH), lambda i: (0, 0))]
        args += [wp]
        out_specs = [bn_d, pl.BlockSpec((BN, 2 * DH), lambda i: (i, 0))]
        out_shape = [jax.ShapeDtypeStruct((N, D), jnp.float32),
                     jax.ShapeDtypeStruct((N, 2 * DH), jnp.float32)]
    return pl.pallas_call(
        _node_body,
        grid=(g,),
        in_specs=in_specs,
        out_specs=out_specs,
        out_shape=out_shape,
    )(*args)


def kernel(x, edge_index, edge_attr, W_enc, b_enc, W_e, b_e, W_n, b_n,
           ln_ng, ln_nb, ln_eg, ln_eb):
    src_flat = edge_index[0]
    dst_flat = edge_index[1]
    dst = dst_flat.reshape(NROW, IW)
    pad_idx = (jnp.arange(EPAD - E, dtype=jnp.int32) * 13) % N
    src_p = jnp.concatenate([src_flat, pad_idx]).reshape(EPAD // IW, IW)
    dst_p = jnp.concatenate([dst_flat, pad_idx]).reshape(EPAD // IW, IW)

    W_enc_s, W_enc_t, W_enc_e = W_enc[:D], W_enc[D:2 * D], W_enc[2 * D:]
    We_s, We_t, We_e = W_e[:D], W_e[D:2 * D], W_e[2 * D:]
    Wn_n, Wn_a = W_n[:D], W_n[D:]
    b_enc2 = b_enc.reshape(1, DH)
    b_e2 = b_e.reshape(1, DH)
    b_n2 = b_n.reshape(1, D)
    ln_eg2, ln_eb2 = ln_eg.reshape(1, DH), ln_eb.reshape(1, DH)
    ln_ng2, ln_nb2 = ln_ng.reshape(1, D), ln_nb.reshape(1, D)

    Ws_cat = jnp.concatenate([W_enc_s, We_s], axis=1)
    Wt_cat = jnp.concatenate([W_enc_t, We_t], axis=1)
    Wcat_e = jnp.concatenate([We_s, We_t], axis=1)

    t_src, t_dst = _proj2(x, Ws_cat, Wt_cat)

    g_both = _gather_dual(t_src, t_dst, src_p, dst_p)

    init_edge, e_new1, e_ln1 = _edge_fused(
        g_both, edge_attr, W_enc_e, b_enc2, We_e, b_e2, ln_eg2, ln_eb2)
    agg1 = _scatter_partial(e_new1, dst)
    node1, p2 = _node_update(x, x, agg1, Wn_n, Wn_a, b_n2,
                             ln_ng2, ln_nb2, Wcat_e)

    g2 = _gather2add(p2, src_p, dst_p)
    e_new2, e_ln2 = _edge_iter(g2, e_ln1, init_edge, We_e, b_e2,
                               ln_eg2, ln_eb2)
    agg2 = _scatter_partial(e_new2, dst)
    node2 = _node_update(node1, x, agg2, Wn_n, Wn_a, b_n2, ln_ng2, ln_nb2)

    return node2, e_ln2


# speedup vs baseline: 4.1116x; 1.1462x over previous
import functools

import jax
import jax.numpy as jnp
from jax import lax
from jax.experimental import pallas as pl
from jax.experimental.pallas import tpu as pltpu
from jax.experimental.pallas import tpu_sc as plsc

N = 10000
E = 320000
D = 128
DH = 64

NC = 2
NS = 16
NW = NC * NS
IW = 128
NROW = E // IW
RPT = N // NS
GRPW = 80
EPAD = GRPW * NW * IW
GBLK = 8
NGBLK = GRPW // GBLK

_SC_PARAMS = pltpu.CompilerParams(use_tc_tiling_on_sc=False)


@functools.lru_cache(maxsize=None)
def _sc_mesh():
    return plsc.VectorSubcoreMesh(core_axis_name="c", subcore_axis_name="s",
                                  num_cores=NC, num_subcores=NS)


def _worker_rows():
    w = lax.axis_index("s") * NC + lax.axis_index("c")
    rs = (w * (NROW // 4)) // (NW // 4)
    re = ((w + 1) * (NROW // 4)) // (NW // 4)
    return rs, re


def _gather_pipelined(ts, td, ia, ib, out, ia_v, ib_v, a0, b0, a1, b1,
                      o0, o1, sema, semb, semo, dual):
    w = lax.axis_index("s") * NC + lax.axis_index("c")
    base = w * GRPW
    bufs = [(a0, b0), (a1, b1)]
    obufs = [o0, o1]

    def block(blk, carry):
        r = base + blk * GBLK
        pltpu.sync_copy(ia.at[pl.ds(r, GBLK)], ia_v)
        pltpu.sync_copy(ib.at[pl.ds(r, GBLK)], ib_v)

        def fire(j):
            aa, bb = bufs[j % 2]
            return (pltpu.async_copy(ts.at[ia_v.at[j]], aa, sema),
                    pltpu.async_copy(td.at[ib_v.at[j]], bb, semb))

        cur = fire(0)
        owaits = [None, None]
        for j in range(GBLK):
            nxt = fire(j + 1) if j < GBLK - 1 else None
            cur[0].wait()
            cur[1].wait()
            aa, bb = bufs[j % 2]
            ob = obufs[j % 2]
            if owaits[j % 2] is not None:
                owaits[j % 2].wait()

            if dual:
                @plsc.parallel_loop(0, IW, step=1, unroll=4)
                def add_row(jj, aa=aa, bb=bb, ob=ob):
                    for k in range(D // 16):
                        s = pl.ds(k * 16, 16)
                        ob[jj, s] = aa[jj, s] + bb[jj, s]
            else:
                @plsc.parallel_loop(0, IW, step=1, unroll=4)
                def add_row(jj, aa=aa, bb=bb, ob=ob):
                    for k in range(DH // 16):
                        ob[jj, pl.ds(k * 16, 16)] = (
                            aa[jj, pl.ds(k * 16, 16)]
                            + bb[jj, pl.ds(DH + k * 16, 16)])
            owaits[j % 2] = pltpu.async_copy(
                ob, out.at[pl.ds((r + j) * IW, IW)], semo)
            cur = nxt
        for ow in owaits:
            if ow is not None:
                ow.wait()
        return carry

    lax.fori_loop(0, NGBLK, block, 0)


_GATHER_SCRATCH = [
    pltpu.VMEM((GBLK, IW), jnp.int32),
    pltpu.VMEM((GBLK, IW), jnp.int32),
    pltpu.VMEM((IW, D), jnp.float32),
    pltpu.VMEM((IW, D), jnp.float32),
    pltpu.VMEM((IW, D), jnp.float32),
    pltpu.VMEM((IW, D), jnp.float32),
    pltpu.VMEM((IW, D), jnp.float32),
    pltpu.VMEM((IW, D), jnp.float32),
    pltpu.SemaphoreType.DMA,
    pltpu.SemaphoreType.DMA,
    pltpu.SemaphoreType.DMA,
]


@functools.lru_cache(maxsize=None)
def _gather_dual_kernel():
    body = functools.partial(_gather_pipelined, dual=True)
    return pl.kernel(
        lambda ts, td, ia, ib, out, *s: body(ts, td, ia, ib, out, *s),
        out_type=jax.ShapeDtypeStruct((EPAD, D), jnp.float32),
        mesh=_sc_mesh(),
        compiler_params=_SC_PARAMS,
        scratch_types=_GATHER_SCRATCH,
    )


@functools.lru_cache(maxsize=None)
def _gather_single_kernel():
    body = functools.partial(_gather_pipelined, dual=False)
    return pl.kernel(
        lambda tp, ia, ib, out, *s: body(tp, tp, ia, ib, out, *s),
        out_type=jax.ShapeDtypeStruct((EPAD, D), jnp.float32),
        mesh=_sc_mesh(),
        compiler_params=_SC_PARAMS,
        scratch_types=_GATHER_SCRATCH,
    )


def _gather_dual(ts, td, src, dst):
    return _gather_dual_kernel()(ts, td, src, dst)


def _gather2add(p, src, dst):
    return _gather_single_kernel()(p, src, dst)


STG = 125


def _scat_body(rows_hbm, idx_hbm, out, acc_sh, stage, idx0, idx1,
               rows0, rows1, semi0, semi1, semr0, semr1):
    cid = lax.axis_index("c")
    sid = lax.axis_index("s")

    @plsc.parallel_loop(0, STG, step=1, unroll=4)
    def zrow(j):
        for k in range(D // 16):
            stage[j, pl.ds(k * 16, 16)] = jnp.zeros((16,), jnp.float32)
    for i in range(RPT // STG):
        pltpu.sync_copy(stage, acc_sh.at[pl.ds(sid * RPT + i * STG, STG)])
    plsc.subcore_barrier()

    rs, re = _worker_rows()
    cnt = re - rs
    sets = [(idx0, rows0, semi0, semr0), (idx1, rows1, semi1, semr1)]

    def fire(r, s):
        idxv, rowsv, si, sr = sets[s]
        r = lax.min(r, re - 1)
        pltpu.async_copy(idx_hbm.at[pl.ds(r, 1)], idxv, si)
        pltpu.async_copy(rows_hbm.at[pl.ds(r * IW, IW)], rowsv, sr)

    def drain(s):
        idxv, rowsv, si, sr = sets[s]
        pltpu.make_async_copy(idx_hbm.at[pl.ds(0, 1)], idxv, si).wait()
        pltpu.make_async_copy(rows_hbm.at[pl.ds(0, IW)], rowsv, sr).wait()

    fire(rs, 0)

    def pair(pi, carry):
        r = rs + 2 * pi
        fire(r + 1, 1)
        drain(0)
        pltpu.sync_copy(rows0, acc_sh.at[idx0.at[0]], add=True)
        fire(r + 2, 0)
        drain(1)

        @pl.when(r + 1 < re)
        def _():
            pltpu.sync_copy(rows1, acc_sh.at[idx1.at[0]], add=True)

        return carry

    lax.fori_loop(0, (cnt + 1) // 2, pair, 0)
    drain(0)
    plsc.subcore_barrier()
    for i in range(RPT // STG):
        pltpu.sync_copy(acc_sh.at[pl.ds(sid * RPT + i * STG, STG)], stage)
        pltpu.sync_copy(stage, out.at[cid, pl.ds(sid * RPT + i * STG, STG)])


@functools.lru_cache(maxsize=None)
def _scatter_kernel():
    return pl.kernel(
        _scat_body,
        out_type=jax.ShapeDtypeStruct((NC, N, D), jnp.float32),
        mesh=_sc_mesh(),
        compiler_params=_SC_PARAMS,
        scratch_types=[
            pltpu.VMEM_SHARED((N, D), jnp.float32),
            pltpu.VMEM((STG, D), jnp.float32),
            pltpu.VMEM((1, IW), jnp.int32),
            pltpu.VMEM((1, IW), jnp.int32),
            pltpu.VMEM((IW, D), jnp.float32),
            pltpu.VMEM((IW, D), jnp.float32),
            pltpu.SemaphoreType.DMA,
            pltpu.SemaphoreType.DMA,
            pltpu.SemaphoreType.DMA,
            pltpu.SemaphoreType.DMA,
        ],
    )


def _scatter_partial(rows, dst):
    return _scatter_kernel()(rows, dst)


BN = 2000
BE = 4000


def _dot(a, b):
    return jnp.dot(a, b, preferred_element_type=jnp.float32)


def _ln(v, g, b):
    m = jnp.mean(v, axis=-1, keepdims=True)
    var = jnp.mean((v - m) ** 2, axis=-1, keepdims=True)
    return (v - m) * jax.lax.rsqrt(var + 1e-5) * g + b


def _proj2_body(x_ref, wa, wb, oa, ob):
    xv = x_ref[...]
    oa[...] = _dot(xv, wa[...])
    ob[...] = _dot(xv, wb[...])


def _proj2(x, wa, wb):
    g = N // BN
    spec_w = pl.BlockSpec((D, 2 * DH), lambda i: (0, 0))
    spec_o = pl.BlockSpec((BN, 2 * DH), lambda i: (i, 0))
    return pl.pallas_call(
        _proj2_body,
        grid=(g,),
        in_specs=[pl.BlockSpec((BN, D), lambda i: (i, 0)), spec_w, spec_w],
        out_specs=[spec_o, spec_o],
        out_shape=[jax.ShapeDtypeStruct((N, 2 * DH), jnp.float32)] * 2,
    )(x, wa, wb)


def _pad_msg(enew):
    be = enew.shape[0]
    return jnp.concatenate(
        [enew, jnp.ones((be, 1), jnp.float32),
         jnp.zeros((be, D - DH - 1), jnp.float32)], axis=1)


def _edge_fused_body(g_ref, ea_ref, wenc_ref, benc_ref, we_ref, be_ref,
                     lg_ref, lb_ref, oinit, onew, oln):
    gb = g_ref[...]
    init = jnp.maximum(
        gb[:, :DH] + _dot(ea_ref[...], wenc_ref[...]) + benc_ref[...], 0.0)
    oinit[...] = init
    enew = jnp.maximum(
        gb[:, DH:] + _dot(init, we_ref[...]) + be_ref[...], 0.0)
    onew[...] = _pad_msg(enew)
    oln[...] = _ln(init + enew, lg_ref[...], lb_ref[...])


def _edge_fused(g_both, edge_attr, wenc, benc, we, be, lg, lb):
    de = edge_attr.shape[1]
    bh = pl.BlockSpec((BE, DH), lambda i: (i, 0))
    row_h = pl.BlockSpec((1, DH), lambda i: (0, 0))
    return pl.pallas_call(
        _edge_fused_body,
        grid=(E // BE,),
        in_specs=[
            pl.BlockSpec((BE, D), lambda i: (i, 0)),
            pl.BlockSpec((BE, de), lambda i: (i, 0)),
            pl.BlockSpec((de, DH), lambda i: (0, 0)),
            row_h,
            pl.BlockSpec((DH, DH), lambda i: (0, 0)),
            row_h, row_h, row_h,
        ],
        out_specs=[bh, pl.BlockSpec((BE, D), lambda i: (i, 0)), bh],
        out_shape=[jax.ShapeDtypeStruct((E, DH), jnp.float32),
                   jax.ShapeDtypeStruct((E, D), jnp.float32),
                   jax.ShapeDtypeStruct((E, DH), jnp.float32)],
    )(g_both, edge_attr, wenc, benc, we, be, lg, lb)


def _edge_iter_body(g_ref, ep_ref, einit_ref, w_ref, b_ref, lg_ref, lb_ref,
                    onew_ref, oln_ref):
    enew = jnp.maximum(
        g_ref[:, :DH] + _dot(ep_ref[...], w_ref[...]) + b_ref[...], 0.0)
    onew_ref[...] = _pad_msg(enew)
    oln_ref[...] = _ln(einit_ref[...] + enew, lg_ref[...], lb_ref[...])


def _edge_iter(g, eprev, einit, w, b, lg, lb):
    bh = pl.BlockSpec((BE, DH), lambda i: (i, 0))
    row_h = pl.BlockSpec((1, DH), lambda i: (0, 0))
    return pl.pallas_call(
        _edge_iter_body,
        grid=(E // BE,),
        in_specs=[pl.BlockSpec((BE, D), lambda i: (i, 0)), bh, bh,
                  pl.BlockSpec((DH, DH), lambda i: (0, 0)),
                  row_h, row_h, row_h],
        out_specs=[pl.BlockSpec((BE, D), lambda i: (i, 0)), bh],
        out_shape=[jax.ShapeDtypeStruct((E, D), jnp.float32),
                   jax.ShapeDtypeStruct((E, DH), jnp.float32)],
    )(g, eprev, einit, w, b, lg, lb)


def _node_body(np_ref, x_ref, agg_ref, wn_ref, wa_ref, b_ref,
               lg_ref, lb_ref, *rest):
    agg = agg_ref[0, :, :DH] + agg_ref[1, :, :DH]
    c = agg_ref[0, :, DH:DH + 1] + agg_ref[1, :, DH:DH + 1]
    inv = 1.0 / jnp.maximum(c, 1.0)
    h = jnp.maximum(
        _dot(np_ref[...], wn_ref[...]) + _dot(agg * inv, wa_ref[...])
        + b_ref[...], 0.0)
    node = _ln(x_ref[...] + h, lg_ref[...], lb_ref[...])
    if len(rest) == 1:
        rest[0][...] = node
    else:
        wp_ref, onode, op_ref = rest
        onode[...] = node
        op_ref[...] = _dot(node, wp_ref[...])


def _node_update(nprev, x, agg_parts, wn, wa, b, lg, lb, wp=None):
    g = N // BN
    bn_d = pl.BlockSpec((BN, D), lambda i: (i, 0))
    row_d = pl.BlockSpec((1, D), lambda i: (0, 0))
    in_specs = [bn_d, bn_d,
                pl.BlockSpec((NC, BN, D), lambda i: (0, i, 0)),
                pl.BlockSpec((D, D), lambda i: (0, 0)),
                pl.BlockSpec((DH, D), lambda i: (0, 0)),
                row_d, row_d, row_d]
    args = [nprev, x, agg_parts, wn, wa, b, lg, lb]
    if wp is None:
        out_specs = bn_d
        out_shape = jax.ShapeDtypeStruct((N, D), jnp.float32)
    else:
        in_specs += [pl.BlockSpec((D, 2 * DH), lambda i: (0, 0))]
        args += [wp]
        out_specs = [bn_d, pl.BlockSpec((BN, 2 * DH), lambda i: (i, 0))]
        out_shape = [jax.ShapeDtypeStruct((N, D), jnp.float32),
                     jax.ShapeDtypeStruct((N, 2 * DH), jnp.float32)]
    return pl.pallas_call(
        _node_body,
        grid=(g,),
        in_specs=in_specs,
        out_specs=out_specs,
        out_shape=out_shape,
    )(*args)


def kernel(x, edge_index, edge_attr, W_enc, b_enc, W_e, b_e, W_n, b_n,
           ln_ng, ln_nb, ln_eg, ln_eb):
    src_flat = edge_index[0]
    dst_flat = edge_index[1]
    dst = dst_flat.reshape(NROW, IW)
    pad_idx = (jnp.arange(EPAD - E, dtype=jnp.int32) * 13) % N
    src_p = jnp.concatenate([src_flat, pad_idx]).reshape(EPAD // IW, IW)
    dst_p = jnp.concatenate([dst_flat, pad_idx]).reshape(EPAD // IW, IW)

    W_enc_s, W_enc_t, W_enc_e = W_enc[:D], W_enc[D:2 * D], W_enc[2 * D:]
    We_s, We_t, We_e = W_e[:D], W_e[D:2 * D], W_e[2 * D:]
    Wn_n, Wn_a = W_n[:D], W_n[D:]
    b_enc2 = b_enc.reshape(1, DH)
    b_e2 = b_e.reshape(1, DH)
    b_n2 = b_n.reshape(1, D)
    ln_eg2, ln_eb2 = ln_eg.reshape(1, DH), ln_eb.reshape(1, DH)
    ln_ng2, ln_nb2 = ln_ng.reshape(1, D), ln_nb.reshape(1, D)

    Ws_cat = jnp.concatenate([W_enc_s, We_s], axis=1)
    Wt_cat = jnp.concatenate([W_enc_t, We_t], axis=1)
    Wcat_e = jnp.concatenate([We_s, We_t], axis=1)

    t_src, t_dst = _proj2(x, Ws_cat, Wt_cat)

    g_both = _gather_dual(t_src, t_dst, src_p, dst_p)

    init_edge, e_new1, e_ln1 = _edge_fused(
        g_both, edge_attr, W_enc_e, b_enc2, We_e, b_e2, ln_eg2, ln_eb2)
    agg1 = _scatter_partial(e_new1, dst)
    node1, p2 = _node_update(x, x, agg1, Wn_n, Wn_a, b_n2,
                             ln_ng2, ln_nb2, Wcat_e)

    g2 = _gather2add(p2, src_p, dst_p)
    e_new2, e_ln2 = _edge_iter(g2, e_ln1, init_edge, We_e, b_e2,
                               ln_eg2, ln_eb2)
    agg2 = _scatter_partial(e_new2, dst)
    node2 = _node_update(node1, x, agg2, Wn_n, Wn_a, b_n2, ln_ng2, ln_nb2)

    return node2, e_ln2


# speedup vs baseline: 4.3280x; 1.0526x over previous
import functools

import jax
import jax.numpy as jnp
from jax import lax
from jax.experimental import pallas as pl
from jax.experimental.pallas import tpu as pltpu
from jax.experimental.pallas import tpu_sc as plsc

N = 10000
E = 320000
D = 128
DH = 64

NC = 2
NS = 16
NW = NC * NS
IW = 128
NROW = E // IW
RPT = N // NS
GRPW = 80
EPAD = GRPW * NW * IW
GBLK = 8
NGBLK = GRPW // GBLK

_SC_PARAMS = pltpu.CompilerParams(use_tc_tiling_on_sc=False)


@functools.lru_cache(maxsize=None)
def _sc_mesh():
    return plsc.VectorSubcoreMesh(core_axis_name="c", subcore_axis_name="s",
                                  num_cores=NC, num_subcores=NS)


def _worker_rows():
    w = lax.axis_index("s") * NC + lax.axis_index("c")
    rs = (w * (NROW // 4)) // (NW // 4)
    re = ((w + 1) * (NROW // 4)) // (NW // 4)
    return rs, re


def _gather_pipelined(ts, td, ia, ib, out, ia_v, ib_v, a0, b0, a1, b1,
                      o0, o1, sema, semb, semo, dual):
    w = lax.axis_index("s") * NC + lax.axis_index("c")
    base = w * GRPW
    bufs = [(a0, b0), (a1, b1)]
    obufs = [o0, o1]

    def block(blk, carry):
        r = base + blk * GBLK
        pltpu.sync_copy(ia.at[pl.ds(r, GBLK)], ia_v)
        pltpu.sync_copy(ib.at[pl.ds(r, GBLK)], ib_v)

        def fire(j):
            aa, bb = bufs[j % 2]
            return (pltpu.async_copy(ts.at[ia_v.at[j]], aa, sema),
                    pltpu.async_copy(td.at[ib_v.at[j]], bb, semb))

        cur = fire(0)
        owaits = [None, None]
        for j in range(GBLK):
            nxt = fire(j + 1) if j < GBLK - 1 else None
            cur[0].wait()
            cur[1].wait()
            aa, bb = bufs[j % 2]
            ob = obufs[j % 2]
            if owaits[j % 2] is not None:
                owaits[j % 2].wait()

            if dual:
                @plsc.parallel_loop(0, IW, step=1, unroll=8)
                def add_row(jj, aa=aa, bb=bb, ob=ob):
                    for k in range(D // 16):
                        s = pl.ds(k * 16, 16)
                        ob[jj, s] = aa[jj, s] + bb[jj, s]
            else:
                @plsc.parallel_loop(0, IW, step=1, unroll=8)
                def add_row(jj, aa=aa, bb=bb, ob=ob):
                    for k in range(DH // 16):
                        s = pl.ds(k * 16, 16)
                        ob[jj, s] = aa[jj, s] + bb[jj, s]
            owaits[j % 2] = pltpu.async_copy(
                ob, out.at[pl.ds((r + j) * IW, IW)], semo)
            cur = nxt
        for ow in owaits:
            if ow is not None:
                ow.wait()
        return carry

    lax.fori_loop(0, NGBLK, block, 0)


def _gather_scratch(buf_cols):
    return [
        pltpu.VMEM((GBLK, IW), jnp.int32),
        pltpu.VMEM((GBLK, IW), jnp.int32),
        pltpu.VMEM((IW, buf_cols), jnp.float32),
        pltpu.VMEM((IW, buf_cols), jnp.float32),
        pltpu.VMEM((IW, buf_cols), jnp.float32),
        pltpu.VMEM((IW, buf_cols), jnp.float32),
        pltpu.VMEM((IW, D), jnp.float32),
        pltpu.VMEM((IW, D), jnp.float32),
        pltpu.SemaphoreType.DMA,
        pltpu.SemaphoreType.DMA,
        pltpu.SemaphoreType.DMA,
    ]


@functools.lru_cache(maxsize=None)
def _gather_dual_kernel():
    body = functools.partial(_gather_pipelined, dual=True)
    return pl.kernel(
        lambda ts, td, ia, ib, out, *s: body(ts, td, ia, ib, out, *s),
        out_type=jax.ShapeDtypeStruct((EPAD, D), jnp.float32),
        mesh=_sc_mesh(),
        compiler_params=_SC_PARAMS,
        scratch_types=_gather_scratch(D),
    )


@functools.lru_cache(maxsize=None)
def _gather_single_kernel():
    body = functools.partial(_gather_pipelined, dual=False)
    return pl.kernel(
        lambda tp, ia, ib, out, *s: body(tp, tp, ia, ib, out, *s),
        out_type=jax.ShapeDtypeStruct((EPAD, D), jnp.float32),
        mesh=_sc_mesh(),
        compiler_params=_SC_PARAMS,
        scratch_types=_gather_scratch(DH),
    )


def _gather_dual(ts, td, src, dst):
    return _gather_dual_kernel()(ts, td, src, dst)


def _gather2add(p, src2, dst2):
    return _gather_single_kernel()(p.reshape(2 * N, DH), src2, dst2)


STG = 125


def _scat_body(rows_hbm, idx_hbm, out, acc_sh, stage, idx0, idx1,
               rows0, rows1, semi0, semi1, semr0, semr1):
    cid = lax.axis_index("c")
    sid = lax.axis_index("s")

    @plsc.parallel_loop(0, STG, step=1, unroll=4)
    def zrow(j):
        for k in range(D // 16):
            stage[j, pl.ds(k * 16, 16)] = jnp.zeros((16,), jnp.float32)
    for i in range(RPT // STG):
        pltpu.sync_copy(stage, acc_sh.at[pl.ds(sid * RPT + i * STG, STG)])
    plsc.subcore_barrier()

    rs, re = _worker_rows()
    cnt = re - rs
    sets = [(idx0, rows0, semi0, semr0), (idx1, rows1, semi1, semr1)]

    def fire(r, s):
        idxv, rowsv, si, sr = sets[s]
        r = lax.min(r, re - 1)
        pltpu.async_copy(idx_hbm.at[pl.ds(r, 1)], idxv, si)
        pltpu.async_copy(rows_hbm.at[pl.ds(r * IW, IW)], rowsv, sr)

    def drain(s):
        idxv, rowsv, si, sr = sets[s]
        pltpu.make_async_copy(idx_hbm.at[pl.ds(0, 1)], idxv, si).wait()
        pltpu.make_async_copy(rows_hbm.at[pl.ds(0, IW)], rowsv, sr).wait()

    fire(rs, 0)

    def pair(pi, carry):
        r = rs + 2 * pi
        fire(r + 1, 1)
        drain(0)
        pltpu.sync_copy(rows0, acc_sh.at[idx0.at[0]], add=True)
        fire(r + 2, 0)
        drain(1)

        @pl.when(r + 1 < re)
        def _():
            pltpu.sync_copy(rows1, acc_sh.at[idx1.at[0]], add=True)

        return carry

    lax.fori_loop(0, (cnt + 1) // 2, pair, 0)
    drain(0)
    plsc.subcore_barrier()
    for i in range(RPT // STG):
        pltpu.sync_copy(acc_sh.at[pl.ds(sid * RPT + i * STG, STG)], stage)
        pltpu.sync_copy(stage, out.at[cid, pl.ds(sid * RPT + i * STG, STG)])


@functools.lru_cache(maxsize=None)
def _scatter_kernel():
    return pl.kernel(
        _scat_body,
        out_type=jax.ShapeDtypeStruct((NC, N, D), jnp.float32),
        mesh=_sc_mesh(),
        compiler_params=_SC_PARAMS,
        scratch_types=[
            pltpu.VMEM_SHARED((N, D), jnp.float32),
            pltpu.VMEM((STG, D), jnp.float32),
            pltpu.VMEM((1, IW), jnp.int32),
            pltpu.VMEM((1, IW), jnp.int32),
            pltpu.VMEM((IW, D), jnp.float32),
            pltpu.VMEM((IW, D), jnp.float32),
            pltpu.SemaphoreType.DMA,
            pltpu.SemaphoreType.DMA,
            pltpu.SemaphoreType.DMA,
            pltpu.SemaphoreType.DMA,
        ],
    )


def _scatter_partial(rows, dst):
    return _scatter_kernel()(rows, dst)


BN = 2000
BE = 4000


def _dot(a, b):
    return jnp.dot(a, b, preferred_element_type=jnp.float32)


def _ln(v, g, b):
    m = jnp.mean(v, axis=-1, keepdims=True)
    var = jnp.mean((v - m) ** 2, axis=-1, keepdims=True)
    return (v - m) * jax.lax.rsqrt(var + 1e-5) * g + b


def _proj2_body(x_ref, wa, wb, oa, ob):
    xv = x_ref[...]
    oa[...] = _dot(xv, wa[...])
    ob[...] = _dot(xv, wb[...])


def _proj2(x, wa, wb):
    g = N // BN
    spec_w = pl.BlockSpec((D, 2 * DH), lambda i: (0, 0))
    spec_o = pl.BlockSpec((BN, 2 * DH), lambda i: (i, 0))
    return pl.pallas_call(
        _proj2_body,
        grid=(g,),
        in_specs=[pl.BlockSpec((BN, D), lambda i: (i, 0)), spec_w, spec_w],
        out_specs=[spec_o, spec_o],
        out_shape=[jax.ShapeDtypeStruct((N, 2 * DH), jnp.float32)] * 2,
    )(x, wa, wb)


def _pad_msg(enew):
    be = enew.shape[0]
    return jnp.concatenate(
        [enew, jnp.ones((be, 1), jnp.float32),
         jnp.zeros((be, D - DH - 1), jnp.float32)], axis=1)


def _edge_fused_body(g_ref, ea_ref, wenc_ref, benc_ref, we_ref, be_ref,
                     lg_ref, lb_ref, oinit, onew, oln):
    gb = g_ref[...]
    init = jnp.maximum(
        gb[:, :DH] + _dot(ea_ref[...], wenc_ref[...]) + benc_ref[...], 0.0)
    oinit[...] = init
    enew = jnp.maximum(
        gb[:, DH:] + _dot(init, we_ref[...]) + be_ref[...], 0.0)
    onew[...] = _pad_msg(enew)
    oln[...] = _ln(init + enew, lg_ref[...], lb_ref[...])


def _edge_fused(g_both, edge_attr, wenc, benc, we, be, lg, lb):
    de = edge_attr.shape[1]
    bh = pl.BlockSpec((BE, DH), lambda i: (i, 0))
    row_h = pl.BlockSpec((1, DH), lambda i: (0, 0))
    return pl.pallas_call(
        _edge_fused_body,
        grid=(E // BE,),
        in_specs=[
            pl.BlockSpec((BE, D), lambda i: (i, 0)),
            pl.BlockSpec((BE, de), lambda i: (i, 0)),
            pl.BlockSpec((de, DH), lambda i: (0, 0)),
            row_h,
            pl.BlockSpec((DH, DH), lambda i: (0, 0)),
            row_h, row_h, row_h,
        ],
        out_specs=[bh, pl.BlockSpec((BE, D), lambda i: (i, 0)), bh],
        out_shape=[jax.ShapeDtypeStruct((E, DH), jnp.float32),
                   jax.ShapeDtypeStruct((E, D), jnp.float32),
                   jax.ShapeDtypeStruct((E, DH), jnp.float32)],
    )(g_both, edge_attr, wenc, benc, we, be, lg, lb)


def _edge_iter_body(g_ref, ep_ref, einit_ref, w_ref, b_ref, lg_ref, lb_ref,
                    onew_ref, oln_ref):
    enew = jnp.maximum(
        g_ref[:, :DH] + _dot(ep_ref[...], w_ref[...]) + b_ref[...], 0.0)
    onew_ref[...] = _pad_msg(enew)
    oln_ref[...] = _ln(einit_ref[...] + enew, lg_ref[...], lb_ref[...])


def _edge_iter(g, eprev, einit, w, b, lg, lb):
    bh = pl.BlockSpec((BE, DH), lambda i: (i, 0))
    row_h = pl.BlockSpec((1, DH), lambda i: (0, 0))
    return pl.pallas_call(
        _edge_iter_body,
        grid=(E // BE,),
        in_specs=[pl.BlockSpec((BE, D), lambda i: (i, 0)), bh, bh,
                  pl.BlockSpec((DH, DH), lambda i: (0, 0)),
                  row_h, row_h, row_h],
        out_specs=[pl.BlockSpec((BE, D), lambda i: (i, 0)), bh],
        out_shape=[jax.ShapeDtypeStruct((E, D), jnp.float32),
                   jax.ShapeDtypeStruct((E, DH), jnp.float32)],
    )(g, eprev, einit, w, b, lg, lb)


def _node_body(np_ref, x_ref, agg_ref, wn_ref, wa_ref, b_ref,
               lg_ref, lb_ref, *rest):
    agg = agg_ref[0, :, :DH] + agg_ref[1, :, :DH]
    c = agg_ref[0, :, DH:DH + 1] + agg_ref[1, :, DH:DH + 1]
    inv = 1.0 / jnp.maximum(c, 1.0)
    h = jnp.maximum(
        _dot(np_ref[...], wn_ref[...]) + _dot(agg * inv, wa_ref[...])
        + b_ref[...], 0.0)
    node = _ln(x_ref[...] + h, lg_ref[...], lb_ref[...])
    if len(rest) == 1:
        rest[0][...] = node
    else:
        wp_ref, onode, op_ref = rest
        onode[...] = node
        op_ref[...] = _dot(node, wp_ref[...])


def _node_update(nprev, x, agg_parts, wn, wa, b, lg, lb, wp=None):
    g = N // BN
    bn_d = pl.BlockSpec((BN, D), lambda i: (i, 0))
    row_d = pl.BlockSpec((1, D), lambda i: (0, 0))
    in_specs = [bn_d, bn_d,
                pl.BlockSpec((NC, BN, D), lambda i: (0, i, 0)),
                pl.BlockSpec((D, D), lambda i: (0, 0)),
                pl.BlockSpec((DH, D), lambda i: (0, 0)),
                row_d, row_d, row_d]
    args = [nprev, x, agg_parts, wn, wa, b, lg, lb]
    if wp is None:
        out_specs = bn_d
        out_shape = jax.ShapeDtypeStruct((N, D), jnp.float32)
    else:
        in_specs += [pl.BlockSpec((D, 2 * DH), lambda i: (0, 0))]
        args += [wp]
        out_specs = [bn_d, pl.BlockSpec((BN, 2 * DH), lambda i: (i, 0))]
        out_shape = [jax.ShapeDtypeStruct((N, D), jnp.float32),
                     jax.ShapeDtypeStruct((N, 2 * DH), jnp.float32)]
    return pl.pallas_call(
        _node_body,
        grid=(g,),
        in_specs=in_specs,
        out_specs=out_specs,
        out_shape=out_shape,
    )(*args)


def kernel(x, edge_index, edge_attr, W_enc, b_enc, W_e, b_e, W_n, b_n,
           ln_ng, ln_nb, ln_eg, ln_eb):
    src_flat = edge_index[0]
    dst_flat = edge_index[1]
    dst = dst_flat.reshape(NROW, IW)
    pad_idx = (jnp.arange(EPAD - E, dtype=jnp.int32) * 13) % N
    src_p = jnp.concatenate([src_flat, pad_idx]).reshape(EPAD // IW, IW)
    dst_p = jnp.concatenate([dst_flat, pad_idx]).reshape(EPAD // IW, IW)

    W_enc_s, W_enc_t, W_enc_e = W_enc[:D], W_enc[D:2 * D], W_enc[2 * D:]
    We_s, We_t, We_e = W_e[:D], W_e[D:2 * D], W_e[2 * D:]
    Wn_n, Wn_a = W_n[:D], W_n[D:]
    b_enc2 = b_enc.reshape(1, DH)
    b_e2 = b_e.reshape(1, DH)
    b_n2 = b_n.reshape(1, D)
    ln_eg2, ln_eb2 = ln_eg.reshape(1, DH), ln_eb.reshape(1, DH)
    ln_ng2, ln_nb2 = ln_ng.reshape(1, D), ln_nb.reshape(1, D)

    Ws_cat = jnp.concatenate([W_enc_s, We_s], axis=1)
    Wt_cat = jnp.concatenate([W_enc_t, We_t], axis=1)
    Wcat_e = jnp.concatenate([We_s, We_t], axis=1)

    t_src, t_dst = _proj2(x, Ws_cat, Wt_cat)

    g_both = _gather_dual(t_src, t_dst, src_p, dst_p)

    init_edge, e_new1, e_ln1 = _edge_fused(
        g_both, edge_attr, W_enc_e, b_enc2, We_e, b_e2, ln_eg2, ln_eb2)
    agg1 = _scatter_partial(e_new1, dst)
    node1, p2 = _node_update(x, x, agg1, Wn_n, Wn_a, b_n2,
                             ln_ng2, ln_nb2, Wcat_e)

    g2 = _gather2add(p2, src_p * 2, dst_p * 2 + 1)
    e_new2, e_ln2 = _edge_iter(g2, e_ln1, init_edge, We_e, b_e2,
                               ln_eg2, ln_eb2)
    agg2 = _scatter_partial(e_new2, dst)
    node2 = _node_update(node1, x, agg2, Wn_n, Wn_a, b_n2, ln_ng2, ln_nb2)

    return node2, e_ln2


# speedup vs baseline: 4.4044x; 1.0177x over previous
import functools

import jax
import jax.numpy as jnp
from jax import lax
from jax.experimental import pallas as pl
from jax.experimental.pallas import tpu as pltpu
from jax.experimental.pallas import tpu_sc as plsc

N = 10000
E = 320000
D = 128
DH = 64

NC = 2
NS = 16
NW = NC * NS
IW = 128
NROW = E // IW
RPT = N // NS
GRPW = 80
EPAD = GRPW * NW * IW
GBLK = 8
NGBLK = GRPW // GBLK

_SC_PARAMS = pltpu.CompilerParams(use_tc_tiling_on_sc=False)


@functools.lru_cache(maxsize=None)
def _sc_mesh():
    return plsc.VectorSubcoreMesh(core_axis_name="c", subcore_axis_name="s",
                                  num_cores=NC, num_subcores=NS)


def _worker_rows():
    w = lax.axis_index("s") * NC + lax.axis_index("c")
    rs = (w * (NROW // 4)) // (NW // 4)
    re = ((w + 1) * (NROW // 4)) // (NW // 4)
    return rs, re


def _gather_pipelined(ts, td, ia, ib, out, ia_v, ib_v, a0, b0, a1, b1,
                      o0, o1, sema, semb, semo, dual):
    w = lax.axis_index("s") * NC + lax.axis_index("c")
    base = w * GRPW
    bufs = [(a0, b0), (a1, b1)]
    obufs = [o0, o1]

    def block(blk, carry):
        r = base + blk * GBLK
        pltpu.sync_copy(ia.at[pl.ds(r, GBLK)], ia_v)
        pltpu.sync_copy(ib.at[pl.ds(r, GBLK)], ib_v)

        def fire(j):
            aa, bb = bufs[j % 2]
            return (pltpu.async_copy(ts.at[ia_v.at[j]], aa, sema),
                    pltpu.async_copy(td.at[ib_v.at[j]], bb, semb))

        cur = fire(0)
        owaits = [None, None]
        for j in range(GBLK):
            nxt = fire(j + 1) if j < GBLK - 1 else None
            cur[0].wait()
            cur[1].wait()
            aa, bb = bufs[j % 2]
            ob = obufs[j % 2]
            if owaits[j % 2] is not None:
                owaits[j % 2].wait()

            if dual:
                @plsc.parallel_loop(0, IW, step=1, unroll=8)
                def add_row(jj, aa=aa, bb=bb, ob=ob):
                    for k in range(D // 16):
                        s = pl.ds(k * 16, 16)
                        ob[jj, s] = aa[jj, s] + bb[jj, s]
            else:
                @plsc.parallel_loop(0, IW, step=1, unroll=8)
                def add_row(jj, aa=aa, bb=bb, ob=ob):
                    for k in range(DH // 16):
                        s = pl.ds(k * 16, 16)
                        ob[jj, s] = aa[jj, s] + bb[jj, s]
            if dual:
                dst_slc = out.at[pl.ds((r + j) * IW, IW)]
            else:
                dst_slc = out.at[pl.ds((r + j) * IW, IW), pl.ds(0, DH)]
            owaits[j % 2] = pltpu.async_copy(ob, dst_slc, semo)
            cur = nxt
        for ow in owaits:
            if ow is not None:
                ow.wait()
        return carry

    lax.fori_loop(0, NGBLK, block, 0)


def _gather_scratch(buf_cols):
    return [
        pltpu.VMEM((GBLK, IW), jnp.int32),
        pltpu.VMEM((GBLK, IW), jnp.int32),
        pltpu.VMEM((IW, buf_cols), jnp.float32),
        pltpu.VMEM((IW, buf_cols), jnp.float32),
        pltpu.VMEM((IW, buf_cols), jnp.float32),
        pltpu.VMEM((IW, buf_cols), jnp.float32),
        pltpu.VMEM((IW, buf_cols), jnp.float32),
        pltpu.VMEM((IW, buf_cols), jnp.float32),
        pltpu.SemaphoreType.DMA,
        pltpu.SemaphoreType.DMA,
        pltpu.SemaphoreType.DMA,
    ]


@functools.lru_cache(maxsize=None)
def _gather_dual_kernel():
    body = functools.partial(_gather_pipelined, dual=True)
    return pl.kernel(
        lambda ts, td, ia, ib, out, *s: body(ts, td, ia, ib, out, *s),
        out_type=jax.ShapeDtypeStruct((EPAD, D), jnp.float32),
        mesh=_sc_mesh(),
        compiler_params=_SC_PARAMS,
        scratch_types=_gather_scratch(D),
    )


@functools.lru_cache(maxsize=None)
def _gather_single_kernel():
    body = functools.partial(_gather_pipelined, dual=False)
    return pl.kernel(
        lambda tp, ia, ib, out, *s: body(tp, tp, ia, ib, out, *s),
        out_type=jax.ShapeDtypeStruct((EPAD, D), jnp.float32),
        mesh=_sc_mesh(),
        compiler_params=_SC_PARAMS,
        scratch_types=_gather_scratch(DH),
    )


def _gather_dual(ts, td, src, dst):
    return _gather_dual_kernel()(ts, td, src, dst)


def _gather2add(p, src2, dst2):
    return _gather_single_kernel()(p.reshape(2 * N, DH), src2, dst2)


STG = 125


def _scat_body(rows_hbm, idx_hbm, out, acc_sh, stage, idx0, idx1,
               rows0, rows1, semi0, semi1, semr0, semr1):
    cid = lax.axis_index("c")
    sid = lax.axis_index("s")

    @plsc.parallel_loop(0, STG, step=1, unroll=4)
    def zrow(j):
        for k in range(D // 16):
            stage[j, pl.ds(k * 16, 16)] = jnp.zeros((16,), jnp.float32)
    for i in range(RPT // STG):
        pltpu.sync_copy(stage, acc_sh.at[pl.ds(sid * RPT + i * STG, STG)])
    plsc.subcore_barrier()

    rs, re = _worker_rows()
    cnt = re - rs
    sets = [(idx0, rows0, semi0, semr0), (idx1, rows1, semi1, semr1)]

    def fire(r, s):
        idxv, rowsv, si, sr = sets[s]
        r = lax.min(r, re - 1)
        pltpu.async_copy(idx_hbm.at[pl.ds(r, 1)], idxv, si)
        pltpu.async_copy(rows_hbm.at[pl.ds(r * IW, IW)], rowsv, sr)

    def drain(s):
        idxv, rowsv, si, sr = sets[s]
        pltpu.make_async_copy(idx_hbm.at[pl.ds(0, 1)], idxv, si).wait()
        pltpu.make_async_copy(rows_hbm.at[pl.ds(0, IW)], rowsv, sr).wait()

    fire(rs, 0)

    def pair(pi, carry):
        r = rs + 2 * pi
        fire(r + 1, 1)
        drain(0)
        pltpu.sync_copy(rows0, acc_sh.at[idx0.at[0]], add=True)
        fire(r + 2, 0)
        drain(1)

        @pl.when(r + 1 < re)
        def _():
            pltpu.sync_copy(rows1, acc_sh.at[idx1.at[0]], add=True)

        return carry

    lax.fori_loop(0, (cnt + 1) // 2, pair, 0)
    drain(0)
    plsc.subcore_barrier()
    for i in range(RPT // STG):
        pltpu.sync_copy(acc_sh.at[pl.ds(sid * RPT + i * STG, STG)], stage)
        pltpu.sync_copy(stage, out.at[cid, pl.ds(sid * RPT + i * STG, STG)])


@functools.lru_cache(maxsize=None)
def _scatter_kernel():
    return pl.kernel(
        _scat_body,
        out_type=jax.ShapeDtypeStruct((NC, N, D), jnp.float32),
        mesh=_sc_mesh(),
        compiler_params=_SC_PARAMS,
        scratch_types=[
            pltpu.VMEM_SHARED((N, D), jnp.float32),
            pltpu.VMEM((STG, D), jnp.float32),
            pltpu.VMEM((1, IW), jnp.int32),
            pltpu.VMEM((1, IW), jnp.int32),
            pltpu.VMEM((IW, D), jnp.float32),
            pltpu.VMEM((IW, D), jnp.float32),
            pltpu.SemaphoreType.DMA,
            pltpu.SemaphoreType.DMA,
            pltpu.SemaphoreType.DMA,
            pltpu.SemaphoreType.DMA,
        ],
    )


def _scatter_partial(rows, dst):
    return _scatter_kernel()(rows, dst)


BN = 2000
BE = 4000


def _dot(a, b):
    return jnp.dot(a, b, preferred_element_type=jnp.float32)


def _ln(v, g, b):
    m = jnp.mean(v, axis=-1, keepdims=True)
    var = jnp.mean((v - m) ** 2, axis=-1, keepdims=True)
    return (v - m) * jax.lax.rsqrt(var + 1e-5) * g + b


def _proj2_body(x_ref, wa, wb, oa, ob):
    xv = x_ref[...]
    oa[...] = _dot(xv, wa[...])
    ob[...] = _dot(xv, wb[...])


def _proj2(x, wa, wb):
    g = N // BN
    spec_w = pl.BlockSpec((D, 2 * DH), lambda i: (0, 0))
    spec_o = pl.BlockSpec((BN, 2 * DH), lambda i: (i, 0))
    return pl.pallas_call(
        _proj2_body,
        grid=(g,),
        in_specs=[pl.BlockSpec((BN, D), lambda i: (i, 0)), spec_w, spec_w],
        out_specs=[spec_o, spec_o],
        out_shape=[jax.ShapeDtypeStruct((N, 2 * DH), jnp.float32)] * 2,
    )(x, wa, wb)


def _pad_msg(enew):
    be = enew.shape[0]
    return jnp.concatenate(
        [enew, jnp.ones((be, 1), jnp.float32),
         jnp.zeros((be, D - DH - 1), jnp.float32)], axis=1)


def _edge_fused_body(g_ref, ea_ref, wenc_ref, benc_ref, we_ref, be_ref,
                     lg_ref, lb_ref, oinit, onew, oln):
    gb = g_ref[...]
    init = jnp.maximum(
        gb[:, :DH] + _dot(ea_ref[...], wenc_ref[...]) + benc_ref[...], 0.0)
    oinit[...] = init
    enew = jnp.maximum(
        gb[:, DH:] + _dot(init, we_ref[...]) + be_ref[...], 0.0)
    onew[...] = _pad_msg(enew)
    oln[...] = _ln(init + enew, lg_ref[...], lb_ref[...])


def _edge_fused(g_both, edge_attr, wenc, benc, we, be, lg, lb):
    de = edge_attr.shape[1]
    bh = pl.BlockSpec((BE, DH), lambda i: (i, 0))
    row_h = pl.BlockSpec((1, DH), lambda i: (0, 0))
    return pl.pallas_call(
        _edge_fused_body,
        grid=(E // BE,),
        in_specs=[
            pl.BlockSpec((BE, D), lambda i: (i, 0)),
            pl.BlockSpec((BE, de), lambda i: (i, 0)),
            pl.BlockSpec((de, DH), lambda i: (0, 0)),
            row_h,
            pl.BlockSpec((DH, DH), lambda i: (0, 0)),
            row_h, row_h, row_h,
        ],
        out_specs=[bh, pl.BlockSpec((BE, D), lambda i: (i, 0)), bh],
        out_shape=[jax.ShapeDtypeStruct((E, DH), jnp.float32),
                   jax.ShapeDtypeStruct((E, D), jnp.float32),
                   jax.ShapeDtypeStruct((E, DH), jnp.float32)],
    )(g_both, edge_attr, wenc, benc, we, be, lg, lb)


def _edge_iter_body(g_ref, ep_ref, einit_ref, w_ref, b_ref, lg_ref, lb_ref,
                    onew_ref, oln_ref):
    enew = jnp.maximum(
        g_ref[:, :DH] + _dot(ep_ref[...], w_ref[...]) + b_ref[...], 0.0)
    onew_ref[...] = _pad_msg(enew)
    oln_ref[...] = _ln(einit_ref[...] + enew, lg_ref[...], lb_ref[...])


def _edge_iter(g, eprev, einit, w, b, lg, lb):
    bh = pl.BlockSpec((BE, DH), lambda i: (i, 0))
    row_h = pl.BlockSpec((1, DH), lambda i: (0, 0))
    return pl.pallas_call(
        _edge_iter_body,
        grid=(E // BE,),
        in_specs=[pl.BlockSpec((BE, D), lambda i: (i, 0)), bh, bh,
                  pl.BlockSpec((DH, DH), lambda i: (0, 0)),
                  row_h, row_h, row_h],
        out_specs=[pl.BlockSpec((BE, D), lambda i: (i, 0)), bh],
        out_shape=[jax.ShapeDtypeStruct((E, D), jnp.float32),
                   jax.ShapeDtypeStruct((E, DH), jnp.float32)],
    )(g, eprev, einit, w, b, lg, lb)


def _node_body(np_ref, x_ref, agg_ref, wn_ref, wa_ref, b_ref,
               lg_ref, lb_ref, *rest):
    agg = agg_ref[0, :, :DH] + agg_ref[1, :, :DH]
    c = agg_ref[0, :, DH:DH + 1] + agg_ref[1, :, DH:DH + 1]
    inv = 1.0 / jnp.maximum(c, 1.0)
    h = jnp.maximum(
        _dot(np_ref[...], wn_ref[...]) + _dot(agg * inv, wa_ref[...])
        + b_ref[...], 0.0)
    node = _ln(x_ref[...] + h, lg_ref[...], lb_ref[...])
    if len(rest) == 1:
        rest[0][...] = node
    else:
        wp_ref, onode, op_ref = rest
        onode[...] = node
        op_ref[...] = _dot(node, wp_ref[...])


def _node_update(nprev, x, agg_parts, wn, wa, b, lg, lb, wp=None):
    g = N // BN
    bn_d = pl.BlockSpec((BN, D), lambda i: (i, 0))
    row_d = pl.BlockSpec((1, D), lambda i: (0, 0))
    in_specs = [bn_d, bn_d,
                pl.BlockSpec((NC, BN, D), lambda i: (0, i, 0)),
                pl.BlockSpec((D, D), lambda i: (0, 0)),
                pl.BlockSpec((DH, D), lambda i: (0, 0)),
                row_d, row_d, row_d]
    args = [nprev, x, agg_parts, wn, wa, b, lg, lb]
    if wp is None:
        out_specs = bn_d
        out_shape = jax.ShapeDtypeStruct((N, D), jnp.float32)
    else:
        in_specs += [pl.BlockSpec((D, 2 * DH), lambda i: (0, 0))]
        args += [wp]
        out_specs = [bn_d, pl.BlockSpec((BN, 2 * DH), lambda i: (i, 0))]
        out_shape = [jax.ShapeDtypeStruct((N, D), jnp.float32),
                     jax.ShapeDtypeStruct((N, 2 * DH), jnp.float32)]
    return pl.pallas_call(
        _node_body,
        grid=(g,),
        in_specs=in_specs,
        out_specs=out_specs,
        out_shape=out_shape,
    )(*args)


def kernel(x, edge_index, edge_attr, W_enc, b_enc, W_e, b_e, W_n, b_n,
           ln_ng, ln_nb, ln_eg, ln_eb):
    src_flat = edge_index[0]
    dst_flat = edge_index[1]
    dst = dst_flat.reshape(NROW, IW)
    pad_idx = (jnp.arange(EPAD - E, dtype=jnp.int32) * 13) % N
    src_p = jnp.concatenate([src_flat, pad_idx]).reshape(EPAD // IW, IW)
    dst_p = jnp.concatenate([dst_flat, pad_idx]).reshape(EPAD // IW, IW)

    W_enc_s, W_enc_t, W_enc_e = W_enc[:D], W_enc[D:2 * D], W_enc[2 * D:]
    We_s, We_t, We_e = W_e[:D], W_e[D:2 * D], W_e[2 * D:]
    Wn_n, Wn_a = W_n[:D], W_n[D:]
    b_enc2 = b_enc.reshape(1, DH)
    b_e2 = b_e.reshape(1, DH)
    b_n2 = b_n.reshape(1, D)
    ln_eg2, ln_eb2 = ln_eg.reshape(1, DH), ln_eb.reshape(1, DH)
    ln_ng2, ln_nb2 = ln_ng.reshape(1, D), ln_nb.reshape(1, D)

    Ws_cat = jnp.concatenate([W_enc_s, We_s], axis=1)
    Wt_cat = jnp.concatenate([W_enc_t, We_t], axis=1)
    Wcat_e = jnp.concatenate([We_s, We_t], axis=1)

    t_src, t_dst = _proj2(x, Ws_cat, Wt_cat)

    g_both = _gather_dual(t_src, t_dst, src_p, dst_p)

    init_edge, e_new1, e_ln1 = _edge_fused(
        g_both, edge_attr, W_enc_e, b_enc2, We_e, b_e2, ln_eg2, ln_eb2)
    agg1 = _scatter_partial(e_new1, dst)
    node1, p2 = _node_update(x, x, agg1, Wn_n, Wn_a, b_n2,
                             ln_ng2, ln_nb2, Wcat_e)

    g2 = _gather2add(p2, src_p * 2, dst_p * 2 + 1)
    e_new2, e_ln2 = _edge_iter(g2, e_ln1, init_edge, We_e, b_e2,
                               ln_eg2, ln_eb2)
    agg2 = _scatter_partial(e_new2, dst)
    node2 = _node_update(node1, x, agg2, Wn_n, Wn_a, b_n2, ln_ng2, ln_nb2)

    return node2, e_ln2


# speedup vs baseline: 4.6833x; 1.0633x over previous
import functools

import jax
import jax.numpy as jnp
from jax import lax
from jax.experimental import pallas as pl
from jax.experimental.pallas import tpu as pltpu
from jax.experimental.pallas import tpu_sc as plsc

N = 10000
E = 320000
D = 128
DH = 64

NC = 2
NS = 16
NW = NC * NS
IW = 128
NROW = E // IW
RPT = N // NS
GRPW = 80
EPAD = GRPW * NW * IW
GBLK = 8
NGBLK = GRPW // GBLK

_SC_PARAMS = pltpu.CompilerParams(use_tc_tiling_on_sc=False)


@functools.lru_cache(maxsize=None)
def _sc_mesh():
    return plsc.VectorSubcoreMesh(core_axis_name="c", subcore_axis_name="s",
                                  num_cores=NC, num_subcores=NS)


def _worker_rows():
    w = lax.axis_index("s") * NC + lax.axis_index("c")
    rs = (w * (NROW // 4)) // (NW // 4)
    re = ((w + 1) * (NROW // 4)) // (NW // 4)
    return rs, re


def _gather_pipelined(ts, td, ia, ib, out, ia_v, ib_v, a0, b0, a1, b1,
                      o0, o1, sema, semb, semo, dual):
    w = lax.axis_index("s") * NC + lax.axis_index("c")
    base = w * GRPW
    bufs = [(a0, b0), (a1, b1)]
    obufs = [o0, o1]

    def block(blk, carry):
        r = base + blk * GBLK
        pltpu.sync_copy(ia.at[pl.ds(r, GBLK)], ia_v)
        pltpu.sync_copy(ib.at[pl.ds(r, GBLK)], ib_v)

        def fire(j):
            aa, bb = bufs[j % 2]
            return (pltpu.async_copy(ts.at[ia_v.at[j]], aa, sema),
                    pltpu.async_copy(td.at[ib_v.at[j]], bb, semb))

        cur = fire(0)
        owaits = [None, None]
        for j in range(GBLK):
            nxt = fire(j + 1) if j < GBLK - 1 else None
            cur[0].wait()
            cur[1].wait()
            aa, bb = bufs[j % 2]
            ob = obufs[j % 2]
            if owaits[j % 2] is not None:
                owaits[j % 2].wait()

            if dual:
                @plsc.parallel_loop(0, IW, step=1, unroll=8)
                def add_row(jj, aa=aa, bb=bb, ob=ob):
                    for k in range(D // 16):
                        s = pl.ds(k * 16, 16)
                        ob[jj, s] = aa[jj, s] + bb[jj, s]
            else:
                @plsc.parallel_loop(0, IW, step=1, unroll=8)
                def add_row(jj, aa=aa, bb=bb, ob=ob):
                    for k in range(DH // 16):
                        s = pl.ds(k * 16, 16)
                        ob[jj, s] = aa[jj, s] + bb[jj, s]
            if dual:
                dst_slc = out.at[pl.ds((r + j) * IW, IW)]
            else:
                dst_slc = out.at[pl.ds((r + j) * IW, IW), pl.ds(0, DH)]
            owaits[j % 2] = pltpu.async_copy(ob, dst_slc, semo)
            cur = nxt
        for ow in owaits:
            if ow is not None:
                ow.wait()
        return carry

    lax.fori_loop(0, NGBLK, block, 0)


def _gather_scratch(buf_cols):
    return [
        pltpu.VMEM((GBLK, IW), jnp.int32),
        pltpu.VMEM((GBLK, IW), jnp.int32),
        pltpu.VMEM((IW, buf_cols), jnp.float32),
        pltpu.VMEM((IW, buf_cols), jnp.float32),
        pltpu.VMEM((IW, buf_cols), jnp.float32),
        pltpu.VMEM((IW, buf_cols), jnp.float32),
        pltpu.VMEM((IW, buf_cols), jnp.float32),
        pltpu.VMEM((IW, buf_cols), jnp.float32),
        pltpu.SemaphoreType.DMA,
        pltpu.SemaphoreType.DMA,
        pltpu.SemaphoreType.DMA,
    ]


@functools.lru_cache(maxsize=None)
def _gather_dual_kernel():
    body = functools.partial(_gather_pipelined, dual=True)
    return pl.kernel(
        lambda ts, td, ia, ib, out, *s: body(ts, td, ia, ib, out, *s),
        out_type=jax.ShapeDtypeStruct((EPAD, D), jnp.float32),
        mesh=_sc_mesh(),
        compiler_params=_SC_PARAMS,
        scratch_types=_gather_scratch(D),
    )


@functools.lru_cache(maxsize=None)
def _gather_single_kernel():
    body = functools.partial(_gather_pipelined, dual=False)
    return pl.kernel(
        lambda tp, ia, ib, out, *s: body(tp, tp, ia, ib, out, *s),
        out_type=jax.ShapeDtypeStruct((EPAD, D), jnp.float32),
        mesh=_sc_mesh(),
        compiler_params=_SC_PARAMS,
        scratch_types=_gather_scratch(DH),
    )


def _gather_dual(ts, td, src, dst):
    return _gather_dual_kernel()(ts, td, src, dst)


def _gather2add(p, src2, dst2):
    return _gather_single_kernel()(p.reshape(2 * N, DH), src2, dst2)


STG = 125


def _scat_body(rows_hbm, idx_hbm, out, acc_sh, stage, idx0, idx1,
               rows0, rows1, semi0, semi1, semr0, semr1):
    cid = lax.axis_index("c")
    sid = lax.axis_index("s")

    @plsc.parallel_loop(0, STG, step=1, unroll=4)
    def zrow(j):
        for k in range(D // 16):
            stage[j, pl.ds(k * 16, 16)] = jnp.zeros((16,), jnp.float32)
    for i in range(RPT // STG):
        pltpu.sync_copy(stage, acc_sh.at[pl.ds(sid * RPT + i * STG, STG)])
    plsc.subcore_barrier()

    rs, re = _worker_rows()
    cnt = re - rs
    sets = [(idx0, rows0, semi0, semr0), (idx1, rows1, semi1, semr1)]

    def fire(r, s):
        idxv, rowsv, si, sr = sets[s]
        r = lax.min(r, re - 1)
        pltpu.async_copy(idx_hbm.at[pl.ds(r, 1)], idxv, si)
        pltpu.async_copy(rows_hbm.at[pl.ds(r * IW, IW)], rowsv, sr)

    def drain(s):
        idxv, rowsv, si, sr = sets[s]
        pltpu.make_async_copy(idx_hbm.at[pl.ds(0, 1)], idxv, si).wait()
        pltpu.make_async_copy(rows_hbm.at[pl.ds(0, IW)], rowsv, sr).wait()

    fire(rs, 0)

    def pair(pi, carry):
        r = rs + 2 * pi
        fire(r + 1, 1)
        drain(0)
        pltpu.sync_copy(rows0, acc_sh.at[idx0.at[0]], add=True)
        fire(r + 2, 0)
        drain(1)

        @pl.when(r + 1 < re)
        def _():
            pltpu.sync_copy(rows1, acc_sh.at[idx1.at[0]], add=True)

        return carry

    lax.fori_loop(0, (cnt + 1) // 2, pair, 0)
    drain(0)
    plsc.subcore_barrier()
    for i in range(RPT // STG):
        pltpu.sync_copy(acc_sh.at[pl.ds(sid * RPT + i * STG, STG)], stage)
        pltpu.sync_copy(stage, out.at[cid, pl.ds(sid * RPT + i * STG, STG)])


@functools.lru_cache(maxsize=None)
def _scatter_kernel():
    return pl.kernel(
        _scat_body,
        out_type=jax.ShapeDtypeStruct((NC, N, D), jnp.float32),
        mesh=_sc_mesh(),
        compiler_params=_SC_PARAMS,
        scratch_types=[
            pltpu.VMEM_SHARED((N, D), jnp.float32),
            pltpu.VMEM((STG, D), jnp.float32),
            pltpu.VMEM((1, IW), jnp.int32),
            pltpu.VMEM((1, IW), jnp.int32),
            pltpu.VMEM((IW, D), jnp.float32),
            pltpu.VMEM((IW, D), jnp.float32),
            pltpu.SemaphoreType.DMA,
            pltpu.SemaphoreType.DMA,
            pltpu.SemaphoreType.DMA,
            pltpu.SemaphoreType.DMA,
        ],
    )


def _scatter_partial(rows, dst):
    return _scatter_kernel()(rows, dst)


BN = 2000
BE = 4000


def _dot(a, b):
    return jnp.dot(a, b, preferred_element_type=jnp.float32)


def _ln(v, g, b):
    m = jnp.mean(v, axis=-1, keepdims=True)
    var = jnp.mean((v - m) ** 2, axis=-1, keepdims=True)
    return (v - m) * jax.lax.rsqrt(var + 1e-5) * g + b


def _proj2_body(x_ref, wa, wb, oa, ob):
    xv = x_ref[...]
    oa[...] = _dot(xv, wa[...])
    ob[...] = _dot(xv, wb[...])


def _proj2(x, wa, wb):
    g = N // BN
    spec_w = pl.BlockSpec((D, 2 * DH), lambda i: (0, 0))
    spec_o = pl.BlockSpec((BN, 2 * DH), lambda i: (i, 0))
    return pl.pallas_call(
        _proj2_body,
        grid=(g,),
        in_specs=[pl.BlockSpec((BN, D), lambda i: (i, 0)), spec_w, spec_w],
        out_specs=[spec_o, spec_o],
        out_shape=[jax.ShapeDtypeStruct((N, 2 * DH), jnp.float32)] * 2,
    )(x, wa, wb)


def _pad_msg(enew):
    be = enew.shape[0]
    return jnp.concatenate(
        [enew, jnp.ones((be, 1), jnp.float32),
         jnp.zeros((be, D - DH - 1), jnp.float32)], axis=1)


def _edge_fused_body(g_ref, ea_ref, wenc_ref, benc_ref, we_ref, be_ref,
                     lg_ref, lb_ref, oinit, onew, oln):
    gb = g_ref[...]
    init = jnp.maximum(
        gb[:, :DH] + _dot(ea_ref[...], wenc_ref[...]) + benc_ref[...], 0.0)
    oinit[...] = init.astype(jnp.bfloat16)
    enew = jnp.maximum(
        gb[:, DH:] + _dot(init, we_ref[...]) + be_ref[...], 0.0)
    onew[...] = _pad_msg(enew)
    oln[...] = _ln(init + enew, lg_ref[...], lb_ref[...]).astype(jnp.bfloat16)


def _edge_fused(g_both, edge_attr, wenc, benc, we, be, lg, lb):
    de = edge_attr.shape[1]
    bh = pl.BlockSpec((BE, DH), lambda i: (i, 0))
    row_h = pl.BlockSpec((1, DH), lambda i: (0, 0))
    return pl.pallas_call(
        _edge_fused_body,
        grid=(E // BE,),
        in_specs=[
            pl.BlockSpec((BE, D), lambda i: (i, 0)),
            pl.BlockSpec((BE, de), lambda i: (i, 0)),
            pl.BlockSpec((de, DH), lambda i: (0, 0)),
            row_h,
            pl.BlockSpec((DH, DH), lambda i: (0, 0)),
            row_h, row_h, row_h,
        ],
        out_specs=[bh, pl.BlockSpec((BE, D), lambda i: (i, 0)), bh],
        out_shape=[jax.ShapeDtypeStruct((E, DH), jnp.bfloat16),
                   jax.ShapeDtypeStruct((E, D), jnp.float32),
                   jax.ShapeDtypeStruct((E, DH), jnp.bfloat16)],
    )(g_both, edge_attr, wenc, benc, we, be, lg, lb)


def _edge_iter_body(g_ref, ep_ref, einit_ref, w_ref, b_ref, lg_ref, lb_ref,
                    onew_ref, oln_ref):
    ep = ep_ref[...].astype(jnp.float32)
    einit = einit_ref[...].astype(jnp.float32)
    enew = jnp.maximum(
        g_ref[:, :DH] + _dot(ep, w_ref[...]) + b_ref[...], 0.0)
    onew_ref[...] = _pad_msg(enew)
    oln_ref[...] = _ln(einit + enew, lg_ref[...], lb_ref[...])


def _edge_iter(g, eprev, einit, w, b, lg, lb):
    bh = pl.BlockSpec((BE, DH), lambda i: (i, 0))
    row_h = pl.BlockSpec((1, DH), lambda i: (0, 0))
    return pl.pallas_call(
        _edge_iter_body,
        grid=(E // BE,),
        in_specs=[pl.BlockSpec((BE, D), lambda i: (i, 0)), bh, bh,
                  pl.BlockSpec((DH, DH), lambda i: (0, 0)),
                  row_h, row_h, row_h],
        out_specs=[pl.BlockSpec((BE, D), lambda i: (i, 0)), bh],
        out_shape=[jax.ShapeDtypeStruct((E, D), jnp.float32),
                   jax.ShapeDtypeStruct((E, DH), jnp.float32)],
    )(g, eprev, einit, w, b, lg, lb)


def _node_body(np_ref, x_ref, agg_ref, wn_ref, wa_ref, b_ref,
               lg_ref, lb_ref, *rest):
    agg = agg_ref[0, :, :DH] + agg_ref[1, :, :DH]
    c = agg_ref[0, :, DH:DH + 1] + agg_ref[1, :, DH:DH + 1]
    inv = 1.0 / jnp.maximum(c, 1.0)
    h = jnp.maximum(
        _dot(np_ref[...], wn_ref[...]) + _dot(agg * inv, wa_ref[...])
        + b_ref[...], 0.0)
    node = _ln(x_ref[...] + h, lg_ref[...], lb_ref[...])
    if len(rest) == 1:
        rest[0][...] = node
    else:
        wp_ref, onode, op_ref = rest
        onode[...] = node
        op_ref[...] = _dot(node, wp_ref[...])


def _node_update(nprev, x, agg_parts, wn, wa, b, lg, lb, wp=None):
    g = N // BN
    bn_d = pl.BlockSpec((BN, D), lambda i: (i, 0))
    row_d = pl.BlockSpec((1, D), lambda i: (0, 0))
    in_specs = [bn_d, bn_d,
                pl.BlockSpec((NC, BN, D), lambda i: (0, i, 0)),
                pl.BlockSpec((D, D), lambda i: (0, 0)),
                pl.BlockSpec((DH, D), lambda i: (0, 0)),
                row_d, row_d, row_d]
    args = [nprev, x, agg_parts, wn, wa, b, lg, lb]
    if wp is None:
        out_specs = bn_d
        out_shape = jax.ShapeDtypeStruct((N, D), jnp.float32)
    else:
        in_specs += [pl.BlockSpec((D, 2 * DH), lambda i: (0, 0))]
        args += [wp]
        out_specs = [bn_d, pl.BlockSpec((BN, 2 * DH), lambda i: (i, 0))]
        out_shape = [jax.ShapeDtypeStruct((N, D), jnp.float32),
                     jax.ShapeDtypeStruct((N, 2 * DH), jnp.float32)]
    return pl.pallas_call(
        _node_body,
        grid=(g,),
        in_specs=in_specs,
        out_specs=out_specs,
        out_shape=out_shape,
    )(*args)


def kernel(x, edge_index, edge_attr, W_enc, b_enc, W_e, b_e, W_n, b_n,
           ln_ng, ln_nb, ln_eg, ln_eb):
    src_flat = edge_index[0]
    dst_flat = edge_index[1]
    dst = dst_flat.reshape(NROW, IW)
    pad_idx = (jnp.arange(EPAD - E, dtype=jnp.int32) * 13) % N
    src_p = jnp.concatenate([src_flat, pad_idx]).reshape(EPAD // IW, IW)
    dst_p = jnp.concatenate([dst_flat, pad_idx]).reshape(EPAD // IW, IW)

    W_enc_s, W_enc_t, W_enc_e = W_enc[:D], W_enc[D:2 * D], W_enc[2 * D:]
    We_s, We_t, We_e = W_e[:D], W_e[D:2 * D], W_e[2 * D:]
    Wn_n, Wn_a = W_n[:D], W_n[D:]
    b_enc2 = b_enc.reshape(1, DH)
    b_e2 = b_e.reshape(1, DH)
    b_n2 = b_n.reshape(1, D)
    ln_eg2, ln_eb2 = ln_eg.reshape(1, DH), ln_eb.reshape(1, DH)
    ln_ng2, ln_nb2 = ln_ng.reshape(1, D), ln_nb.reshape(1, D)

    Ws_cat = jnp.concatenate([W_enc_s, We_s], axis=1)
    Wt_cat = jnp.concatenate([W_enc_t, We_t], axis=1)
    Wcat_e = jnp.concatenate([We_s, We_t], axis=1)

    t_src, t_dst = _proj2(x, Ws_cat, Wt_cat)

    g_both = _gather_dual(t_src, t_dst, src_p, dst_p)

    init_edge, e_new1, e_ln1 = _edge_fused(
        g_both, edge_attr, W_enc_e, b_enc2, We_e, b_e2, ln_eg2, ln_eb2)
    agg1 = _scatter_partial(e_new1, dst)
    node1, p2 = _node_update(x, x, agg1, Wn_n, Wn_a, b_n2,
                             ln_ng2, ln_nb2, Wcat_e)

    g2 = _gather2add(p2, src_p * 2, dst_p * 2 + 1)
    e_new2, e_ln2 = _edge_iter(g2, e_ln1, init_edge, We_e, b_e2,
                               ln_eg2, ln_eb2)
    agg2 = _scatter_partial(e_new2, dst)
    node2 = _node_update(node1, x, agg2, Wn_n, Wn_a, b_n2, ln_ng2, ln_nb2)

    return node2, e_ln2


# speedup vs baseline: 4.8359x; 1.0326x over previous
import functools

import jax
import jax.numpy as jnp
from jax import lax
from jax.experimental import pallas as pl
from jax.experimental.pallas import tpu as pltpu
from jax.experimental.pallas import tpu_sc as plsc

N = 10000
E = 320000
D = 128
DH = 64

NC = 2
NS = 16
NW = NC * NS
IW = 128
NROW = E // IW
RPT = N // NS
GRPW = 80
EPAD = GRPW * NW * IW
GBLK = 8
NGBLK = GRPW // GBLK

_SC_PARAMS = pltpu.CompilerParams(use_tc_tiling_on_sc=False)


@functools.lru_cache(maxsize=None)
def _sc_mesh():
    return plsc.VectorSubcoreMesh(core_axis_name="c", subcore_axis_name="s",
                                  num_cores=NC, num_subcores=NS)


def _worker_rows():
    w = lax.axis_index("s") * NC + lax.axis_index("c")
    rs = (w * (NROW // 4)) // (NW // 4)
    re = ((w + 1) * (NROW // 4)) // (NW // 4)
    return rs, re


def _gather_pipelined(ts, td, ia, ib, out, ia_v, ib_v, a0, b0, a1, b1,
                      o0, o1, sema, semb, semo, dual):
    w = lax.axis_index("s") * NC + lax.axis_index("c")
    base = w * GRPW
    bufs = [(a0, b0), (a1, b1)]
    obufs = [o0, o1]

    def block(blk, carry):
        r = base + blk * GBLK
        pltpu.sync_copy(ia.at[pl.ds(r, GBLK)], ia_v)
        pltpu.sync_copy(ib.at[pl.ds(r, GBLK)], ib_v)

        def fire(j):
            aa, bb = bufs[j % 2]
            return (pltpu.async_copy(ts.at[ia_v.at[j]], aa, sema),
                    pltpu.async_copy(td.at[ib_v.at[j]], bb, semb))

        cur = fire(0)
        owaits = [None, None]
        for j in range(GBLK):
            nxt = fire(j + 1) if j < GBLK - 1 else None
            cur[0].wait()
            cur[1].wait()
            aa, bb = bufs[j % 2]
            ob = obufs[j % 2]
            if owaits[j % 2] is not None:
                owaits[j % 2].wait()

            if dual:
                @plsc.parallel_loop(0, IW, step=1, unroll=8)
                def add_row(jj, aa=aa, bb=bb, ob=ob):
                    for k in range(D // 16):
                        s = pl.ds(k * 16, 16)
                        ob[jj, s] = aa[jj, s] + bb[jj, s]
            else:
                @plsc.parallel_loop(0, IW, step=1, unroll=8)
                def add_row(jj, aa=aa, bb=bb, ob=ob):
                    for k in range(DH // 16):
                        s = pl.ds(k * 16, 16)
                        ob[jj, s] = aa[jj, s] + bb[jj, s]
            if dual:
                dst_slc = out.at[pl.ds((r + j) * IW, IW)]
            else:
                dst_slc = out.at[pl.ds((r + j) * IW, IW), pl.ds(0, DH)]
            owaits[j % 2] = pltpu.async_copy(ob, dst_slc, semo)
            cur = nxt
        for ow in owaits:
            if ow is not None:
                ow.wait()
        return carry

    lax.fori_loop(0, NGBLK, block, 0)


def _gather_scratch(buf_cols):
    return [
        pltpu.VMEM((GBLK, IW), jnp.int32),
        pltpu.VMEM((GBLK, IW), jnp.int32),
        pltpu.VMEM((IW, buf_cols), jnp.float32),
        pltpu.VMEM((IW, buf_cols), jnp.float32),
        pltpu.VMEM((IW, buf_cols), jnp.float32),
        pltpu.VMEM((IW, buf_cols), jnp.float32),
        pltpu.VMEM((IW, buf_cols), jnp.float32),
        pltpu.VMEM((IW, buf_cols), jnp.float32),
        pltpu.SemaphoreType.DMA,
        pltpu.SemaphoreType.DMA,
        pltpu.SemaphoreType.DMA,
    ]


@functools.lru_cache(maxsize=None)
def _gather_dual_kernel():
    body = functools.partial(_gather_pipelined, dual=True)
    return pl.kernel(
        lambda ts, td, ia, ib, out, *s: body(ts, td, ia, ib, out, *s),
        out_type=jax.ShapeDtypeStruct((EPAD, D), jnp.float32),
        mesh=_sc_mesh(),
        compiler_params=_SC_PARAMS,
        scratch_types=_gather_scratch(D),
    )


@functools.lru_cache(maxsize=None)
def _gather_single_kernel():
    body = functools.partial(_gather_pipelined, dual=False)
    return pl.kernel(
        lambda tp, ia, ib, out, *s: body(tp, tp, ia, ib, out, *s),
        out_type=jax.ShapeDtypeStruct((EPAD, D), jnp.float32),
        mesh=_sc_mesh(),
        compiler_params=_SC_PARAMS,
        scratch_types=_gather_scratch(DH),
    )


def _gather_dual(ts, td, src, dst):
    return _gather_dual_kernel()(ts, td, src, dst)


def _gather2add(p, src2, dst2):
    return _gather_single_kernel()(p.reshape(2 * N, DH), src2, dst2)


STG = 125


def _scat_body(rows_hbm, idx_hbm, out, acc_sh, stage, idx0, idx1,
               rows0, rows1, semi0, semi1, semr0, semr1):
    cid = lax.axis_index("c")
    sid = lax.axis_index("s")

    @plsc.parallel_loop(0, STG, step=1, unroll=4)
    def zrow(j):
        for k in range(D // 16):
            stage[j, pl.ds(k * 16, 16)] = jnp.zeros((16,), jnp.float32)
    for i in range(RPT // STG):
        pltpu.sync_copy(stage, acc_sh.at[pl.ds(sid * RPT + i * STG, STG)])
    plsc.subcore_barrier()

    rs, re = _worker_rows()
    cnt = re - rs
    sets = [(idx0, rows0, semi0, semr0), (idx1, rows1, semi1, semr1)]

    def fire(r, s):
        idxv, rowsv, si, sr = sets[s]
        r = lax.min(r, re - 1)
        pltpu.async_copy(idx_hbm.at[pl.ds(r, 1)], idxv, si)
        pltpu.async_copy(rows_hbm.at[pl.ds(r * IW, IW)], rowsv, sr)

    def drain(s):
        idxv, rowsv, si, sr = sets[s]
        pltpu.make_async_copy(idx_hbm.at[pl.ds(0, 1)], idxv, si).wait()
        pltpu.make_async_copy(rows_hbm.at[pl.ds(0, IW)], rowsv, sr).wait()

    fire(rs, 0)

    def pair(pi, carry):
        r = rs + 2 * pi
        fire(r + 1, 1)
        drain(0)
        pltpu.sync_copy(rows0, acc_sh.at[idx0.at[0]], add=True)
        fire(r + 2, 0)
        drain(1)

        @pl.when(r + 1 < re)
        def _():
            pltpu.sync_copy(rows1, acc_sh.at[idx1.at[0]], add=True)

        return carry

    lax.fori_loop(0, (cnt + 1) // 2, pair, 0)
    drain(0)
    plsc.subcore_barrier()
    for i in range(RPT // STG):
        pltpu.sync_copy(acc_sh.at[pl.ds(sid * RPT + i * STG, STG)], stage)
        pltpu.sync_copy(stage, out.at[cid, pl.ds(sid * RPT + i * STG, STG)])


@functools.lru_cache(maxsize=None)
def _scatter_kernel():
    return pl.kernel(
        _scat_body,
        out_type=jax.ShapeDtypeStruct((NC, N, D), jnp.float32),
        mesh=_sc_mesh(),
        compiler_params=_SC_PARAMS,
        scratch_types=[
            pltpu.VMEM_SHARED((N, D), jnp.float32),
            pltpu.VMEM((STG, D), jnp.float32),
            pltpu.VMEM((1, IW), jnp.int32),
            pltpu.VMEM((1, IW), jnp.int32),
            pltpu.VMEM((IW, D), jnp.float32),
            pltpu.VMEM((IW, D), jnp.float32),
            pltpu.SemaphoreType.DMA,
            pltpu.SemaphoreType.DMA,
            pltpu.SemaphoreType.DMA,
            pltpu.SemaphoreType.DMA,
        ],
    )


def _scatter_partial(rows, dst):
    return _scatter_kernel()(rows, dst)


BN = 2000
BE = 8000


def _dot(a, b):
    return jnp.dot(a, b, preferred_element_type=jnp.float32)


def _ln(v, g, b):
    m = jnp.mean(v, axis=-1, keepdims=True)
    var = jnp.mean((v - m) ** 2, axis=-1, keepdims=True)
    return (v - m) * jax.lax.rsqrt(var + 1e-5) * g + b


def _proj2_body(x_ref, wa, wb, oa, ob):
    xv = x_ref[...]
    oa[...] = _dot(xv, wa[...])
    ob[...] = _dot(xv, wb[...])


def _proj2(x, wa, wb):
    g = N // BN
    spec_w = pl.BlockSpec((D, 2 * DH), lambda i: (0, 0))
    spec_o = pl.BlockSpec((BN, 2 * DH), lambda i: (i, 0))
    return pl.pallas_call(
        _proj2_body,
        grid=(g,),
        in_specs=[pl.BlockSpec((BN, D), lambda i: (i, 0)), spec_w, spec_w],
        out_specs=[spec_o, spec_o],
        out_shape=[jax.ShapeDtypeStruct((N, 2 * DH), jnp.float32)] * 2,
    )(x, wa, wb)


def _pad_msg(enew):
    be = enew.shape[0]
    return jnp.concatenate(
        [enew, jnp.ones((be, 1), jnp.float32),
         jnp.zeros((be, D - DH - 1), jnp.float32)], axis=1)


def _edge_fused_body(g_ref, ea_ref, wenc_ref, benc_ref, we_ref, be_ref,
                     lg_ref, lb_ref, oinit, onew, oln):
    gb = g_ref[...]
    init = jnp.maximum(
        gb[:, :DH] + _dot(ea_ref[...], wenc_ref[...]) + benc_ref[...], 0.0)
    oinit[...] = init.astype(jnp.bfloat16)
    enew = jnp.maximum(
        gb[:, DH:] + _dot(init, we_ref[...]) + be_ref[...], 0.0)
    onew[...] = _pad_msg(enew)
    oln[...] = _ln(init + enew, lg_ref[...], lb_ref[...]).astype(jnp.bfloat16)


def _edge_fused(g_both, edge_attr, wenc, benc, we, be, lg, lb):
    de = edge_attr.shape[1]
    bh = pl.BlockSpec((BE, DH), lambda i: (i, 0))
    row_h = pl.BlockSpec((1, DH), lambda i: (0, 0))
    return pl.pallas_call(
        _edge_fused_body,
        grid=(E // BE,),
        in_specs=[
            pl.BlockSpec((BE, D), lambda i: (i, 0)),
            pl.BlockSpec((BE, de), lambda i: (i, 0)),
            pl.BlockSpec((de, DH), lambda i: (0, 0)),
            row_h,
            pl.BlockSpec((DH, DH), lambda i: (0, 0)),
            row_h, row_h, row_h,
        ],
        out_specs=[bh, pl.BlockSpec((BE, D), lambda i: (i, 0)), bh],
        out_shape=[jax.ShapeDtypeStruct((E, DH), jnp.bfloat16),
                   jax.ShapeDtypeStruct((E, D), jnp.float32),
                   jax.ShapeDtypeStruct((E, DH), jnp.bfloat16)],
    )(g_both, edge_attr, wenc, benc, we, be, lg, lb)


def _edge_iter_body(g_ref, ep_ref, einit_ref, w_ref, b_ref, lg_ref, lb_ref,
                    onew_ref, oln_ref):
    ep = ep_ref[...].astype(jnp.float32)
    einit = einit_ref[...].astype(jnp.float32)
    enew = jnp.maximum(
        g_ref[:, :DH] + _dot(ep, w_ref[...]) + b_ref[...], 0.0)
    onew_ref[...] = _pad_msg(enew)
    oln_ref[...] = _ln(einit + enew, lg_ref[...], lb_ref[...])


def _edge_iter(g, eprev, einit, w, b, lg, lb):
    bh = pl.BlockSpec((BE, DH), lambda i: (i, 0))
    row_h = pl.BlockSpec((1, DH), lambda i: (0, 0))
    return pl.pallas_call(
        _edge_iter_body,
        grid=(E // BE,),
        in_specs=[pl.BlockSpec((BE, D), lambda i: (i, 0)), bh, bh,
                  pl.BlockSpec((DH, DH), lambda i: (0, 0)),
                  row_h, row_h, row_h],
        out_specs=[pl.BlockSpec((BE, D), lambda i: (i, 0)), bh],
        out_shape=[jax.ShapeDtypeStruct((E, D), jnp.float32),
                   jax.ShapeDtypeStruct((E, DH), jnp.float32)],
    )(g, eprev, einit, w, b, lg, lb)


def _node_body(np_ref, x_ref, agg_ref, wn_ref, wa_ref, b_ref,
               lg_ref, lb_ref, *rest):
    agg = agg_ref[0, :, :DH] + agg_ref[1, :, :DH]
    c = agg_ref[0, :, DH:DH + 1] + agg_ref[1, :, DH:DH + 1]
    inv = 1.0 / jnp.maximum(c, 1.0)
    h = jnp.maximum(
        _dot(np_ref[...], wn_ref[...]) + _dot(agg * inv, wa_ref[...])
        + b_ref[...], 0.0)
    node = _ln(x_ref[...] + h, lg_ref[...], lb_ref[...])
    if len(rest) == 1:
        rest[0][...] = node
    else:
        wp_ref, onode, op_ref = rest
        onode[...] = node
        op_ref[...] = _dot(node, wp_ref[...])


def _node_update(nprev, x, agg_parts, wn, wa, b, lg, lb, wp=None):
    g = N // BN
    bn_d = pl.BlockSpec((BN, D), lambda i: (i, 0))
    row_d = pl.BlockSpec((1, D), lambda i: (0, 0))
    in_specs = [bn_d, bn_d,
                pl.BlockSpec((NC, BN, D), lambda i: (0, i, 0)),
                pl.BlockSpec((D, D), lambda i: (0, 0)),
                pl.BlockSpec((DH, D), lambda i: (0, 0)),
                row_d, row_d, row_d]
    args = [nprev, x, agg_parts, wn, wa, b, lg, lb]
    if wp is None:
        out_specs = bn_d
        out_shape = jax.ShapeDtypeStruct((N, D), jnp.float32)
    else:
        in_specs += [pl.BlockSpec((D, 2 * DH), lambda i: (0, 0))]
        args += [wp]
        out_specs = [bn_d, pl.BlockSpec((BN, 2 * DH), lambda i: (i, 0))]
        out_shape = [jax.ShapeDtypeStruct((N, D), jnp.float32),
                     jax.ShapeDtypeStruct((N, 2 * DH), jnp.float32)]
    return pl.pallas_call(
        _node_body,
        grid=(g,),
        in_specs=in_specs,
        out_specs=out_specs,
        out_shape=out_shape,
    )(*args)


def kernel(x, edge_index, edge_attr, W_enc, b_enc, W_e, b_e, W_n, b_n,
           ln_ng, ln_nb, ln_eg, ln_eb):
    src_flat = edge_index[0]
    dst_flat = edge_index[1]
    dst = dst_flat.reshape(NROW, IW)
    pad_idx = (jnp.arange(EPAD - E, dtype=jnp.int32) * 13) % N
    src_p = jnp.concatenate([src_flat, pad_idx]).reshape(EPAD // IW, IW)
    dst_p = jnp.concatenate([dst_flat, pad_idx]).reshape(EPAD // IW, IW)

    W_enc_s, W_enc_t, W_enc_e = W_enc[:D], W_enc[D:2 * D], W_enc[2 * D:]
    We_s, We_t, We_e = W_e[:D], W_e[D:2 * D], W_e[2 * D:]
    Wn_n, Wn_a = W_n[:D], W_n[D:]
    b_enc2 = b_enc.reshape(1, DH)
    b_e2 = b_e.reshape(1, DH)
    b_n2 = b_n.reshape(1, D)
    ln_eg2, ln_eb2 = ln_eg.reshape(1, DH), ln_eb.reshape(1, DH)
    ln_ng2, ln_nb2 = ln_ng.reshape(1, D), ln_nb.reshape(1, D)

    Ws_cat = jnp.concatenate([W_enc_s, We_s], axis=1)
    Wt_cat = jnp.concatenate([W_enc_t, We_t], axis=1)
    Wcat_e = jnp.concatenate([We_s, We_t], axis=1)

    t_src, t_dst = _proj2(x, Ws_cat, Wt_cat)

    g_both = _gather_dual(t_src, t_dst, src_p, dst_p)

    init_edge, e_new1, e_ln1 = _edge_fused(
        g_both, edge_attr, W_enc_e, b_enc2, We_e, b_e2, ln_eg2, ln_eb2)
    agg1 = _scatter_partial(e_new1, dst)
    node1, p2 = _node_update(x, x, agg1, Wn_n, Wn_a, b_n2,
                             ln_ng2, ln_nb2, Wcat_e)

    g2 = _gather2add(p2, src_p * 2, dst_p * 2 + 1)
    e_new2, e_ln2 = _edge_iter(g2, e_ln1, init_edge, We_e, b_e2,
                               ln_eg2, ln_eb2)
    agg2 = _scatter_partial(e_new2, dst)
    node2 = _node_update(node1, x, agg2, Wn_n, Wn_a, b_n2, ln_ng2, ln_nb2)

    return node2, e_ln2
